# baseline (speedup 1.0000x reference)
import functools

import jax
import jax.numpy as jnp
import numpy as np
from jax.experimental import pallas as pl
from jax.experimental.pallas import tpu as pltpu

B = 1
S = 2048
DIM = 768
NH = 12
G = 2
HPG = NH // G
DK = 64
DV = 64
CL = 32
CD = 16
LSEL = 64
NSEL = 8
WIN = 512
TAU = 1.0
HID = DK // 2
C = (S - CL) // CD + 1
CPAD = 128
NB = S // LSEL
QB = 128
NEG = -1e30
SCALE = 1.0 / 8.0

_OFF_Q = 0
_OFF_KS = NH * DK
_OFF_VS = _OFF_KS + G * DK
_OFF_KW = _OFF_VS + G * DV
_OFF_VW = _OFF_KW + G * DK
_OFF_KC = _OFF_VW + G * DV
_OFF_VC = _OFF_KC + G * DK
_PTOT = _OFF_VC + G * DV


def _swap_halves(xx, nheads):
    parts = []
    for h in range(nheads):
        base = h * DK
        parts.append(xx[:, base + DK // 2: base + DK])
        parts.append(xx[:, base: base + DK // 2])
    return jnp.concatenate(parts, axis=1)


RB = 512


def _bf16_dot(a, b, dn):
    return jax.lax.dot_general(a.astype(jnp.bfloat16), b.astype(jnp.bfloat16),
                               dn, preferred_element_type=jnp.float32)


def _proj_kernel(x_ref, w_ref, cosq_ref, sinq_ref, cosk_ref, sink_ref,
                 pool_ref, fc1w_ref, fc1b_ref, fc2w_ref, fc2b_ref,
                 qr_ref, ks_ref, vs_ref, kw_ref, vw_ref, kc_ref, vc_ref,
                 wg_ref):
    step = pl.program_id(0)
    x = x_ref[...]
    p = jax.lax.dot_general(x, w_ref[...], (((1,), (0,)), ((), ())),
                            preferred_element_type=jnp.float32)
    q = p[:, _OFF_Q:_OFF_KS]
    cq = cosq_ref[...]
    sq = sinq_ref[...]
    ck = cosk_ref[...]
    sk = sink_ref[...]
    qr_ref[...] = q * cq + _swap_halves(q, NH) * sq

    ks = p[:, _OFF_KS:_OFF_VS]
    ks_ref[...] = ks * ck + _swap_halves(ks, G) * sk
    vs_ref[...] = p[:, _OFF_VS:_OFF_KW]
    kw = p[:, _OFF_KW:_OFF_VW]
    kw_ref[...] = kw * ck + _swap_halves(kw, G) * sk
    vw_ref[...] = p[:, _OFF_VW:_OFF_KC]
    kcr = p[:, _OFF_KC:_OFF_VC]
    kcr = kcr * ck + _swap_halves(kcr, G) * sk
    pool = pool_ref[...]
    kc_part = jax.lax.dot_general(pool, kcr, (((1,), (0,)), ((), ())),
                                  preferred_element_type=jnp.float32,
                                  precision=jax.lax.Precision.HIGHEST)
    vc_part = jax.lax.dot_general(pool, p[:, _OFF_VC:_PTOT],
                                  (((1,), (0,)), ((), ())),
                                  preferred_element_type=jnp.float32,
                                  precision=jax.lax.Precision.HIGHEST)

    @pl.when(step == 0)
    def _init():
        kc_ref[...] = kc_part
        vc_ref[...] = vc_part

    @pl.when(step > 0)
    def _acc():
        kc_ref[...] += kc_part
        vc_ref[...] += vc_part

    @pl.when(step == pl.num_programs(0) - 1)
    def _finish():
        kc_ref[...] = kc_ref[...] * (1.0 / CL)
        vc_ref[...] = vc_ref[...] * (1.0 / CL)

    fc1w = fc1w_ref[...]
    fc1b = fc1b_ref[...]
    fc2w = fc2w_ref[...]
    fc2b = fc2b_ref[...]
    gate_cols = []
    for g in range(G):
        qg = q[:, g * HPG * DK:(g + 1) * HPG * DK]
        acc = qg[:, 0:DK]
        for h in range(1, HPG):
            acc = acc + qg[:, h * DK:(h + 1) * DK]
        qgp = acc / float(HPG)
        h1 = _bf16_dot(qgp, fc1w, (((1,), (1,)), ((), ()))) + fc1b
        h1 = h1 * jax.nn.sigmoid(h1)
        gl = _bf16_dot(h1, fc2w, (((1,), (1,)), ((), ()))) + fc2b
        x0 = gl[:, 0:1]
        x1 = gl[:, 1:2]
        x2 = gl[:, 2:3]
        mx = jnp.maximum(jnp.maximum(x0, x1), x2)
        mn = jnp.minimum(jnp.minimum(x0, x1), x2)
        mid = x0 + x1 + x2 - mx - mn
        e0 = jnp.exp(x0 - mx)
        e1 = jnp.exp(x1 - mx)
        e2 = jnp.exp(x2 - mx)
        z = e0 + e1 + e2
        peaked = (mx - mid) > 50.0
        a0 = x0 == mx
        a1 = (x1 == mx) & (~a0)
        a2 = (x2 == mx) & (~a0) & (~a1)
        w0 = jnp.where(peaked, a0.astype(jnp.float32), e0 / z)
        w1 = jnp.where(peaked, a1.astype(jnp.float32), e1 / z)
        w2 = jnp.where(peaked, a2.astype(jnp.float32), e2 / z)
        gate_cols += [w0, w1, w2]
    gate_cols.append(jnp.zeros((x.shape[0], 2), jnp.float32))
    wg_ref[...] = jnp.concatenate(gate_cols, axis=1)


def _masked_softmax(scores, mask):
    sm = jnp.where(mask, scores, NEG)
    mx = jnp.max(sm, axis=-1, keepdims=True)
    p = jnp.where(mask, jnp.exp(sm - mx), 0.0)
    denom = jnp.sum(p, axis=-1, keepdims=True)
    return jnp.where(denom > 0.0, p / jnp.where(denom > 0.0, denom, 1.0), 0.0)


def _attn_kernel(qr_ref, wg_ref, ks_ref, vs_ref, kw_ref, vw_ref,
                 kc_ref, vc_ref, ov_ref, exp_ref, wout_ref, out_ref,
                 osc_ref):
    i = pl.program_id(0)
    q0 = i * QB
    t = q0 + jax.lax.broadcasted_iota(jnp.int32, (QB, 1), 0)
    ccol = jax.lax.broadcasted_iota(jnp.int32, (QB, CPAD), 1)
    bcol = jax.lax.broadcasted_iota(jnp.int32, (QB, NB), 1)

    m_cmp = ((ccol * CD + CL) <= (t + 1)) & (ccol < C)
    causal_blk = (bcol * LSEL) <= t
    forced = (bcol == 0) | (bcol == (t // LSEL))

    o_cmp_all = []
    sel_all = []
    gate_all = []
    for g in range(G):
        kcg = kc_ref[:, g * DK:(g + 1) * DK]
        vcg = vc_ref[:, g * DV:(g + 1) * DV]

        o_cmps = []
        psum = None
        for h in range(HPG):
            qh = qr_ref[:, (g * HPG + h) * DK:(g * HPG + h + 1) * DK]
            sc = _bf16_dot(qh, kcg, (((1,), (1,)), ((), ())))
            pc = _masked_softmax(sc * SCALE, m_cmp)
            psum = pc if psum is None else psum + pc
            o_cmps.append(
                _bf16_dot(pc, vcg, (((1,), (0,)), ((), ()))))
        o_cmp_all.append(o_cmps)

        p_slc = _bf16_dot(psum, ov_ref[...], (((1,), (0,)), ((), ())))
        score = jnp.where(causal_blk, p_slc, -1e9) + \
            jnp.where(forced, 1e6, 0.0)

        sel = jnp.zeros((QB, NB), jnp.float32)
        work = score
        for _ in range(NSEL):
            mx = jnp.max(work, axis=-1, keepdims=True)
            cand = jnp.where(work == mx, bcol, NB + 1)
            amin = jnp.min(cand, axis=-1, keepdims=True)
            pick = bcol == amin
            sel = jnp.where(pick, 1.0, sel)
            work = jnp.where(pick, -3e9, work)
        sel_all.append(sel)
        gate_all.append((wg_ref[:, g * 3:g * 3 + 1],
                         wg_ref[:, g * 3 + 1:g * 3 + 2],
                         wg_ref[:, g * 3 + 2:g * 3 + 3]))

    WWIN = WIN + QB
    for variant in range(S // QB // 4):
        @pl.when(i // 4 == variant)
        def _go(variant=variant):
            w = (variant + 1) * 4 * QB
            kcw = jax.lax.broadcasted_iota(jnp.int32, (QB, w), 1)
            causal_w = kcw <= t
            if variant == 0:
                jw = jax.lax.broadcasted_iota(jnp.int32, (QB, WIN), 1)
                win_m = (jw <= t) & (jw > t - WIN)
                wstart = 0
                wlen = WIN
            else:
                wstart = (i - 4) * QB
                jw = wstart + jax.lax.broadcasted_iota(
                    jnp.int32, (QB, WWIN), 1)
                win_m = (jw <= t) & (jw > t - WIN)
                wlen = WWIN
            for g in range(G):
                tok = _bf16_dot(sel_all[g], exp_ref[:, 0:w],
                                (((1,), (0,)), ((), ())))
                sel_mask = (tok > 0.5) & causal_w
                ksg = ks_ref[0:w, g * DK:(g + 1) * DK]
                vsg = vs_ref[0:w, g * DV:(g + 1) * DV]
                kwg = kw_ref[pl.ds(wstart, wlen), g * DK:(g + 1) * DK]
                vwg = vw_ref[pl.ds(wstart, wlen), g * DV:(g + 1) * DV]
                w_cmp, w_sel, w_win = gate_all[g]
                for h in range(HPG):
                    hh = g * HPG + h
                    qh = qr_ref[:, hh * DK:(hh + 1) * DK]
                    ss = _bf16_dot(qh, ksg, (((1,), (1,)), ((), ())))
                    ps = _masked_softmax(ss * SCALE, sel_mask)
                    o_sel = _bf16_dot(ps, vsg, (((1,), (0,)), ((), ())))
                    sw = _bf16_dot(qh, kwg, (((1,), (1,)), ((), ())))
                    pw = _masked_softmax(sw * SCALE, win_m)
                    o_win = _bf16_dot(pw, vwg, (((1,), (0,)), ((), ())))
                    osc_ref[:, hh * DV:(hh + 1) * DV] = (
                        w_cmp * o_cmp_all[g][h] + w_sel * o_sel
                        + w_win * o_win)

    out_ref[...] = _bf16_dot(osc_ref[...], wout_ref[...],
                             (((1,), (0,)), ((), ())))


def _rope_tables():
    pos = jnp.arange(S, dtype=jnp.float32)
    half = DK // 2
    freqs = 1.0 / (10000.0 ** (jnp.arange(half, dtype=jnp.float32) / half))
    ang = pos[:, None] * freqs[None, :]
    cos = jnp.cos(ang)
    sin = jnp.sin(ang)
    cos_h = jnp.concatenate([cos, cos], axis=1)
    sin_h = jnp.concatenate([-sin, sin], axis=1)
    cosq = jnp.tile(cos_h, (1, NH))
    sinq = jnp.tile(sin_h, (1, NH))
    cosk = jnp.tile(cos_h, (1, G))
    sink = jnp.tile(sin_h, (1, G))
    return cosq, sinq, cosk, sink


def _const_tables():
    pool = np.zeros((CPAD, S), dtype=np.float32)
    for c in range(C):
        pool[c, c * CD:c * CD + CL] = 1.0
    pool = jnp.asarray(pool)

    cstart = np.arange(C) * CD
    bstart = np.arange(NB) * LSEL
    ov = np.clip(np.minimum(cstart[:, None] + CL, bstart[None, :] + LSEL)
                 - np.maximum(cstart[:, None], bstart[None, :]),
                 0, None).astype(np.float32) / CL
    ovp = np.zeros((CPAD, NB), dtype=np.float32)
    ovp[:C] = ov
    ovp = jnp.asarray(ovp)

    expm = np.zeros((NB, S), dtype=np.float32)
    for bnum in range(NB):
        expm[bnum, bnum * LSEL:(bnum + 1) * LSEL] = 1.0
    expm = jnp.asarray(expm)
    return pool, ovp, expm


def _run_proj(x2, w_all, fc1_w, fc1_b, fc2_w, fc2_b):
    cosq, sinq, cosk, sink = _rope_tables()
    pool, _, _ = _const_tables()
    fc2w_pad = jnp.zeros((8, HID), jnp.float32).at[:3].set(fc2_w)
    fc2b_pad = jnp.zeros((1, 8), jnp.float32).at[0, :3].set(fc2_b)

    return pl.pallas_call(
        _proj_kernel,
        grid=(S // RB,),
        in_specs=[
            pl.BlockSpec((RB, DIM), lambda i: (i, 0)),
            pl.BlockSpec((DIM, _PTOT), lambda i: (0, 0)),
            pl.BlockSpec((RB, NH * DK), lambda i: (i, 0)),
            pl.BlockSpec((RB, NH * DK), lambda i: (i, 0)),
            pl.BlockSpec((RB, G * DK), lambda i: (i, 0)),
            pl.BlockSpec((RB, G * DK), lambda i: (i, 0)),
            pl.BlockSpec((CPAD, RB), lambda i: (0, i)),
            pl.BlockSpec((HID, DK), lambda i: (0, 0)),
            pl.BlockSpec((1, HID), lambda i: (0, 0)),
            pl.BlockSpec((8, HID), lambda i: (0, 0)),
            pl.BlockSpec((1, 8), lambda i: (0, 0)),
        ],
        out_specs=[
            pl.BlockSpec((RB, NH * DK), lambda i: (i, 0)),
            pl.BlockSpec((RB, G * DK), lambda i: (i, 0)),
            pl.BlockSpec((RB, G * DV), lambda i: (i, 0)),
            pl.BlockSpec((RB, G * DK), lambda i: (i, 0)),
            pl.BlockSpec((RB, G * DV), lambda i: (i, 0)),
            pl.BlockSpec((CPAD, G * DK), lambda i: (0, 0)),
            pl.BlockSpec((CPAD, G * DV), lambda i: (0, 0)),
            pl.BlockSpec((RB, 8), lambda i: (i, 0)),
        ],
        out_shape=[
            jax.ShapeDtypeStruct((S, NH * DK), jnp.float32),
            jax.ShapeDtypeStruct((S, G * DK), jnp.float32),
            jax.ShapeDtypeStruct((S, G * DV), jnp.float32),
            jax.ShapeDtypeStruct((S, G * DK), jnp.float32),
            jax.ShapeDtypeStruct((S, G * DV), jnp.float32),
            jax.ShapeDtypeStruct((CPAD, G * DK), jnp.float32),
            jax.ShapeDtypeStruct((CPAD, G * DV), jnp.float32),
            jax.ShapeDtypeStruct((S, 8), jnp.float32),
        ],
    )(x2.astype(jnp.bfloat16), w_all.astype(jnp.bfloat16),
      cosq, sinq, cosk, sink, pool,
      fc1_w.astype(jnp.bfloat16), fc1_b.reshape(1, HID),
      fc2w_pad.astype(jnp.bfloat16), fc2b_pad)


def _run_attn(qr, wg, ks, vs, kw, vw, kc, vc, W_out):
    _, ovp, expm = _const_tables()
    nsteps = S // QB
    return pl.pallas_call(
        _attn_kernel,
        grid=(nsteps,),
        in_specs=[
            pl.BlockSpec((QB, NH * DK), lambda i: (i, 0)),
            pl.BlockSpec((QB, 8), lambda i: (i, 0)),
            pl.BlockSpec((S, G * DK), lambda i: (0, 0)),
            pl.BlockSpec((S, G * DV), lambda i: (0, 0)),
            pl.BlockSpec((S, G * DK), lambda i: (0, 0)),
            pl.BlockSpec((S, G * DV), lambda i: (0, 0)),
            pl.BlockSpec((CPAD, G * DK), lambda i: (0, 0)),
            pl.BlockSpec((CPAD, G * DV), lambda i: (0, 0)),
            pl.BlockSpec((CPAD, NB), lambda i: (0, 0)),
            pl.BlockSpec((NB, S), lambda i: (0, 0)),
            pl.BlockSpec((NH * DV, DIM), lambda i: (0, 0)),
        ],
        out_specs=pl.BlockSpec((QB, DIM), lambda i: (i, 0)),
        out_shape=jax.ShapeDtypeStruct((S, DIM), jnp.float32),
        scratch_shapes=[pltpu.VMEM((QB, NH * DV), jnp.float32)],
    )(qr.astype(jnp.bfloat16), wg,
      ks.astype(jnp.bfloat16), vs.astype(jnp.bfloat16),
      kw.astype(jnp.bfloat16), vw.astype(jnp.bfloat16),
      kc.astype(jnp.bfloat16), vc.astype(jnp.bfloat16),
      ovp.astype(jnp.bfloat16), expm.astype(jnp.bfloat16),
      W_out.T.astype(jnp.bfloat16))


@functools.partial(jax.jit, static_argnames=())
def kernel(x, W_Q, W_K_sel, W_V_sel, W_K_win, W_V_win, W_K_cmp, W_V_cmp,
           W_out, fc1_w, fc1_b, fc2_w, fc2_b):
    x2 = x.reshape(S, DIM)
    w_all = jnp.concatenate(
        [W_Q, W_K_sel, W_V_sel, W_K_win, W_V_win, W_K_cmp, W_V_cmp],
        axis=0).T
    qr, ks, vs, kw, vw, kc, vc, wg = _run_proj(
        x2, w_all, fc1_w, fc1_b, fc2_w, fc2_b)
    out = _run_attn(qr, wg, ks, vs, kw, vw, kc, vc, W_out)
    return out.reshape(B, S, DIM)


# speedup vs baseline: 1.7345x; 1.7345x over previous
import functools

import jax
import jax.numpy as jnp
import numpy as np
from jax.experimental import pallas as pl
from jax.experimental.pallas import tpu as pltpu

B = 1
S = 2048
DIM = 768
NH = 12
G = 2
HPG = NH // G
DK = 64
DV = 64
CL = 32
CD = 16
LSEL = 64
NSEL = 8
WIN = 512
TAU = 1.0
HID = DK // 2
C = (S - CL) // CD + 1
CPAD = 128
NB = S // LSEL
QB = 128
NEG = -1e30
SCALE = 1.0 / 8.0

_OFF_Q = 0
_OFF_KS = NH * DK
_OFF_VS = _OFF_KS + G * DK
_OFF_KW = _OFF_VS + G * DV
_OFF_VW = _OFF_KW + G * DK
_OFF_KC = _OFF_VW + G * DV
_OFF_VC = _OFF_KC + G * DK
_PTOT = _OFF_VC + G * DV


def _swap_halves(xx, nheads):
    parts = []
    for h in range(nheads):
        base = h * DK
        parts.append(xx[:, base + DK // 2: base + DK])
        parts.append(xx[:, base: base + DK // 2])
    return jnp.concatenate(parts, axis=1)


RB = 512


def _bf16_dot(a, b, dn):
    return jax.lax.dot_general(a.astype(jnp.bfloat16), b.astype(jnp.bfloat16),
                               dn, preferred_element_type=jnp.float32)


def _proj_kernel(x_ref, w_ref, cosq_ref, sinq_ref, cosk_ref, sink_ref,
                 pool_ref, fc1w_ref, fc1b_ref, fc2w_ref, fc2b_ref,
                 qr_ref, ks_ref, vs_ref, kw_ref, vw_ref, kc_ref, vc_ref,
                 wg_ref):
    step = pl.program_id(0)
    x = x_ref[...]
    p = jax.lax.dot_general(x, w_ref[...], (((1,), (0,)), ((), ())),
                            preferred_element_type=jnp.float32)
    q = p[:, _OFF_Q:_OFF_KS]
    cq = cosq_ref[...]
    sq = sinq_ref[...]
    ck = cosk_ref[...]
    sk = sink_ref[...]
    qr_ref[...] = q * cq + _swap_halves(q, NH) * sq

    ks = p[:, _OFF_KS:_OFF_VS]
    ks_ref[...] = ks * ck + _swap_halves(ks, G) * sk
    vs_ref[...] = p[:, _OFF_VS:_OFF_KW]
    kw = p[:, _OFF_KW:_OFF_VW]
    kw_ref[...] = kw * ck + _swap_halves(kw, G) * sk
    vw_ref[...] = p[:, _OFF_VW:_OFF_KC]
    kcr = p[:, _OFF_KC:_OFF_VC]
    kcr = kcr * ck + _swap_halves(kcr, G) * sk
    pool = pool_ref[...]
    kc_part = jax.lax.dot_general(pool, kcr, (((1,), (0,)), ((), ())),
                                  preferred_element_type=jnp.float32,
                                  precision=jax.lax.Precision.HIGHEST)
    vc_part = jax.lax.dot_general(pool, p[:, _OFF_VC:_PTOT],
                                  (((1,), (0,)), ((), ())),
                                  preferred_element_type=jnp.float32,
                                  precision=jax.lax.Precision.HIGHEST)

    @pl.when(step == 0)
    def _init():
        kc_ref[...] = kc_part
        vc_ref[...] = vc_part

    @pl.when(step > 0)
    def _acc():
        kc_ref[...] += kc_part
        vc_ref[...] += vc_part

    @pl.when(step == pl.num_programs(0) - 1)
    def _finish():
        kc_ref[...] = kc_ref[...] * (1.0 / CL)
        vc_ref[...] = vc_ref[...] * (1.0 / CL)

    fc1w = fc1w_ref[...]
    fc1b = fc1b_ref[...]
    fc2w = fc2w_ref[...]
    fc2b = fc2b_ref[...]
    gate_cols = []
    for g in range(G):
        qg = q[:, g * HPG * DK:(g + 1) * HPG * DK]
        acc = qg[:, 0:DK]
        for h in range(1, HPG):
            acc = acc + qg[:, h * DK:(h + 1) * DK]
        qgp = acc / float(HPG)
        h1 = _bf16_dot(qgp, fc1w, (((1,), (1,)), ((), ()))) + fc1b
        h1 = h1 * jax.nn.sigmoid(h1)
        gl = _bf16_dot(h1, fc2w, (((1,), (1,)), ((), ()))) + fc2b
        x0 = gl[:, 0:1]
        x1 = gl[:, 1:2]
        x2 = gl[:, 2:3]
        mx = jnp.maximum(jnp.maximum(x0, x1), x2)
        mn = jnp.minimum(jnp.minimum(x0, x1), x2)
        mid = x0 + x1 + x2 - mx - mn
        e0 = jnp.exp(x0 - mx)
        e1 = jnp.exp(x1 - mx)
        e2 = jnp.exp(x2 - mx)
        z = e0 + e1 + e2
        peaked = (mx - mid) > 50.0
        a0 = x0 == mx
        a1 = (x1 == mx) & (~a0)
        a2 = (x2 == mx) & (~a0) & (~a1)
        w0 = jnp.where(peaked, a0.astype(jnp.float32), e0 / z)
        w1 = jnp.where(peaked, a1.astype(jnp.float32), e1 / z)
        w2 = jnp.where(peaked, a2.astype(jnp.float32), e2 / z)
        gate_cols += [w0, w1, w2]
    gate_cols.append(jnp.zeros((x.shape[0], 2), jnp.float32))
    wg_ref[...] = jnp.concatenate(gate_cols, axis=1)


def _masked_softmax(scores, mask):
    sm = jnp.where(mask, scores, NEG)
    mx = jnp.max(sm, axis=-1, keepdims=True)
    p = jnp.where(mask, jnp.exp(sm - mx), 0.0)
    denom = jnp.sum(p, axis=-1, keepdims=True)
    return jnp.where(denom > 0.0, p / jnp.where(denom > 0.0, denom, 1.0), 0.0)


def _attn_kernel(qr_ref, wg_ref, ks_ref, vs_ref, kw_ref, vw_ref,
                 kc_ref, vc_ref, ov_ref, exp_ref, wout_ref, out_ref,
                 osc_ref):
    i = pl.program_id(0)
    q0 = i * QB
    t = q0 + jax.lax.broadcasted_iota(jnp.int32, (QB, 1), 0)
    ccol = jax.lax.broadcasted_iota(jnp.int32, (QB, CPAD), 1)
    bcol = jax.lax.broadcasted_iota(jnp.int32, (QB, NB), 1)

    m_cmp = ((ccol * CD + CL) <= (t + 1)) & (ccol < C)
    causal_blk = (bcol * LSEL) <= t
    forced = (bcol == 0) | (bcol == (t // LSEL))

    o_cmp_all = []
    sel_all = []
    gate_all = []
    for g in range(G):
        kcg = kc_ref[:, g * DK:(g + 1) * DK]
        vcg = vc_ref[:, g * DV:(g + 1) * DV]

        o_cmps = []
        psum = None
        for h in range(HPG):
            qh = qr_ref[:, (g * HPG + h) * DK:(g * HPG + h + 1) * DK]
            sc = _bf16_dot(qh, kcg, (((1,), (1,)), ((), ())))
            pc = _masked_softmax(sc * SCALE, m_cmp)
            psum = pc if psum is None else psum + pc
            o_cmps.append(
                _bf16_dot(pc, vcg, (((1,), (0,)), ((), ()))))
        o_cmp_all.append(o_cmps)

        p_slc = _bf16_dot(psum, ov_ref[...], (((1,), (0,)), ((), ())))
        score = jnp.where(causal_blk, p_slc, -1e9) + \
            jnp.where(forced, 1e6, 0.0)

        sel = jnp.zeros((QB, NB), jnp.float32)
        work = score
        for _ in range(NSEL):
            mx = jnp.max(work, axis=-1, keepdims=True)
            cand = jnp.where(work == mx, bcol, NB + 1)
            amin = jnp.min(cand, axis=-1, keepdims=True)
            pick = bcol == amin
            sel = jnp.where(pick, 1.0, sel)
            work = jnp.where(pick, -3e9, work)
        sel_all.append(sel)
        gate_all.append((wg_ref[:, g * 3:g * 3 + 1],
                         wg_ref[:, g * 3 + 1:g * 3 + 2],
                         wg_ref[:, g * 3 + 2:g * 3 + 3]))

    WWIN = WIN + QB
    kcol = jax.lax.broadcasted_iota(jnp.int32, (QB, S), 1)
    causal = kcol <= t
    wstart = jnp.maximum(i - 4, 0) * QB
    jw = wstart + jax.lax.broadcasted_iota(jnp.int32, (QB, WWIN), 1)
    win_m = (jw <= t) & (jw > t - WIN)
    o_parts = []
    for g in range(G):
        tok = _bf16_dot(sel_all[g], exp_ref[...], (((1,), (0,)), ((), ())))
        sel_mask = (tok > 0.5) & causal
        ksg = ks_ref[:, g * DK:(g + 1) * DK]
        vsg = vs_ref[:, g * DV:(g + 1) * DV]
        kwg = kw_ref[pl.ds(wstart, WWIN), g * DK:(g + 1) * DK]
        vwg = vw_ref[pl.ds(wstart, WWIN), g * DV:(g + 1) * DV]
        w_cmp, w_sel, w_win = gate_all[g]
        for h in range(HPG):
            hh = g * HPG + h
            qh = qr_ref[:, hh * DK:(hh + 1) * DK]
            ss = _bf16_dot(qh, ksg, (((1,), (1,)), ((), ())))
            ps = _masked_softmax(ss * SCALE, sel_mask)
            o_sel = _bf16_dot(ps, vsg, (((1,), (0,)), ((), ())))
            sw = _bf16_dot(qh, kwg, (((1,), (1,)), ((), ())))
            pw = _masked_softmax(sw * SCALE, win_m)
            o_win = _bf16_dot(pw, vwg, (((1,), (0,)), ((), ())))
            o_parts.append(w_cmp * o_cmp_all[g][h] + w_sel * o_sel
                           + w_win * o_win)

    o_all = jnp.concatenate(o_parts, axis=1)
    out_ref[...] = _bf16_dot(o_all, wout_ref[...], (((1,), (0,)), ((), ())))


def _rope_tables():
    pos = jnp.arange(S, dtype=jnp.float32)
    half = DK // 2
    freqs = 1.0 / (10000.0 ** (jnp.arange(half, dtype=jnp.float32) / half))
    ang = pos[:, None] * freqs[None, :]
    cos = jnp.cos(ang)
    sin = jnp.sin(ang)
    cos_h = jnp.concatenate([cos, cos], axis=1)
    sin_h = jnp.concatenate([-sin, sin], axis=1)
    cosq = jnp.tile(cos_h, (1, NH))
    sinq = jnp.tile(sin_h, (1, NH))
    cosk = jnp.tile(cos_h, (1, G))
    sink = jnp.tile(sin_h, (1, G))
    return cosq, sinq, cosk, sink


def _const_tables():
    pool = np.zeros((CPAD, S), dtype=np.float32)
    for c in range(C):
        pool[c, c * CD:c * CD + CL] = 1.0
    pool = jnp.asarray(pool)

    cstart = np.arange(C) * CD
    bstart = np.arange(NB) * LSEL
    ov = np.clip(np.minimum(cstart[:, None] + CL, bstart[None, :] + LSEL)
                 - np.maximum(cstart[:, None], bstart[None, :]),
                 0, None).astype(np.float32) / CL
    ovp = np.zeros((CPAD, NB), dtype=np.float32)
    ovp[:C] = ov
    ovp = jnp.asarray(ovp)

    expm = np.zeros((NB, S), dtype=np.float32)
    for bnum in range(NB):
        expm[bnum, bnum * LSEL:(bnum + 1) * LSEL] = 1.0
    expm = jnp.asarray(expm)
    return pool, ovp, expm


def _run_proj(x2, w_all, fc1_w, fc1_b, fc2_w, fc2_b):
    cosq, sinq, cosk, sink = _rope_tables()
    pool, _, _ = _const_tables()
    fc2w_pad = jnp.zeros((8, HID), jnp.float32).at[:3].set(fc2_w)
    fc2b_pad = jnp.zeros((1, 8), jnp.float32).at[0, :3].set(fc2_b)

    return pl.pallas_call(
        _proj_kernel,
        grid=(S // RB,),
        in_specs=[
            pl.BlockSpec((RB, DIM), lambda i: (i, 0)),
            pl.BlockSpec((DIM, _PTOT), lambda i: (0, 0)),
            pl.BlockSpec((RB, NH * DK), lambda i: (i, 0)),
            pl.BlockSpec((RB, NH * DK), lambda i: (i, 0)),
            pl.BlockSpec((RB, G * DK), lambda i: (i, 0)),
            pl.BlockSpec((RB, G * DK), lambda i: (i, 0)),
            pl.BlockSpec((CPAD, RB), lambda i: (0, i)),
            pl.BlockSpec((HID, DK), lambda i: (0, 0)),
            pl.BlockSpec((1, HID), lambda i: (0, 0)),
            pl.BlockSpec((8, HID), lambda i: (0, 0)),
            pl.BlockSpec((1, 8), lambda i: (0, 0)),
        ],
        out_specs=[
            pl.BlockSpec((RB, NH * DK), lambda i: (i, 0)),
            pl.BlockSpec((RB, G * DK), lambda i: (i, 0)),
            pl.BlockSpec((RB, G * DV), lambda i: (i, 0)),
            pl.BlockSpec((RB, G * DK), lambda i: (i, 0)),
            pl.BlockSpec((RB, G * DV), lambda i: (i, 0)),
            pl.BlockSpec((CPAD, G * DK), lambda i: (0, 0)),
            pl.BlockSpec((CPAD, G * DV), lambda i: (0, 0)),
            pl.BlockSpec((RB, 8), lambda i: (i, 0)),
        ],
        out_shape=[
            jax.ShapeDtypeStruct((S, NH * DK), jnp.float32),
            jax.ShapeDtypeStruct((S, G * DK), jnp.float32),
            jax.ShapeDtypeStruct((S, G * DV), jnp.float32),
            jax.ShapeDtypeStruct((S, G * DK), jnp.float32),
            jax.ShapeDtypeStruct((S, G * DV), jnp.float32),
            jax.ShapeDtypeStruct((CPAD, G * DK), jnp.float32),
            jax.ShapeDtypeStruct((CPAD, G * DV), jnp.float32),
            jax.ShapeDtypeStruct((S, 8), jnp.float32),
        ],
    )(x2.astype(jnp.bfloat16), w_all.astype(jnp.bfloat16),
      cosq, sinq, cosk, sink, pool,
      fc1_w.astype(jnp.bfloat16), fc1_b.reshape(1, HID),
      fc2w_pad.astype(jnp.bfloat16), fc2b_pad)


def _run_attn(qr, wg, ks, vs, kw, vw, kc, vc, W_out):
    _, ovp, expm = _const_tables()
    nsteps = S // QB
    return pl.pallas_call(
        _attn_kernel,
        grid=(nsteps,),
        in_specs=[
            pl.BlockSpec((QB, NH * DK), lambda i: (i, 0)),
            pl.BlockSpec((QB, 8), lambda i: (i, 0)),
            pl.BlockSpec((S, G * DK), lambda i: (0, 0)),
            pl.BlockSpec((S, G * DV), lambda i: (0, 0)),
            pl.BlockSpec((S, G * DK), lambda i: (0, 0)),
            pl.BlockSpec((S, G * DV), lambda i: (0, 0)),
            pl.BlockSpec((CPAD, G * DK), lambda i: (0, 0)),
            pl.BlockSpec((CPAD, G * DV), lambda i: (0, 0)),
            pl.BlockSpec((CPAD, NB), lambda i: (0, 0)),
            pl.BlockSpec((NB, S), lambda i: (0, 0)),
            pl.BlockSpec((NH * DV, DIM), lambda i: (0, 0)),
        ],
        out_specs=pl.BlockSpec((QB, DIM), lambda i: (i, 0)),
        out_shape=jax.ShapeDtypeStruct((S, DIM), jnp.float32),
        scratch_shapes=[pltpu.VMEM((QB, NH * DV), jnp.float32)],
    )(qr.astype(jnp.bfloat16), wg,
      ks.astype(jnp.bfloat16), vs.astype(jnp.bfloat16),
      kw.astype(jnp.bfloat16), vw.astype(jnp.bfloat16),
      kc.astype(jnp.bfloat16), vc.astype(jnp.bfloat16),
      ovp.astype(jnp.bfloat16), expm.astype(jnp.bfloat16),
      W_out.T.astype(jnp.bfloat16))


@functools.partial(jax.jit, static_argnames=())
def kernel(x, W_Q, W_K_sel, W_V_sel, W_K_win, W_V_win, W_K_cmp, W_V_cmp,
           W_out, fc1_w, fc1_b, fc2_w, fc2_b):
    x2 = x.reshape(S, DIM)
    w_all = jnp.concatenate(
        [W_Q, W_K_sel, W_V_sel, W_K_win, W_V_win, W_K_cmp, W_V_cmp],
        axis=0).T
    qr, ks, vs, kw, vw, kc, vc, wg = _run_proj(
        x2, w_all, fc1_w, fc1_b, fc2_w, fc2_b)
    out = _run_attn(qr, wg, ks, vs, kw, vw, kc, vc, W_out)
    return out.reshape(B, S, DIM)


# speedup vs baseline: 1.8734x; 1.0801x over previous
import functools

import jax
import jax.numpy as jnp
import numpy as np
from jax.experimental import pallas as pl
from jax.experimental.pallas import tpu as pltpu

B = 1
S = 2048
DIM = 768
NH = 12
G = 2
HPG = NH // G
DK = 64
DV = 64
CL = 32
CD = 16
LSEL = 64
NSEL = 8
WIN = 512
TAU = 1.0
HID = DK // 2
C = (S - CL) // CD + 1
CPAD = 128
NB = S // LSEL
QB = 128
NEG = -1e30
SCALE = 1.0 / 8.0

_OFF_Q = 0
_OFF_KS = NH * DK
_OFF_VS = _OFF_KS + G * DK
_OFF_KW = _OFF_VS + G * DV
_OFF_VW = _OFF_KW + G * DK
_OFF_KC = _OFF_VW + G * DV
_OFF_VC = _OFF_KC + G * DK
_PTOT = _OFF_VC + G * DV


def _swap_halves(xx, nheads):
    parts = []
    for h in range(nheads):
        base = h * DK
        parts.append(xx[:, base + DK // 2: base + DK])
        parts.append(xx[:, base: base + DK // 2])
    return jnp.concatenate(parts, axis=1)


RB = 512


def _bf16_dot(a, b, dn):
    return jax.lax.dot_general(a.astype(jnp.bfloat16), b.astype(jnp.bfloat16),
                               dn, preferred_element_type=jnp.float32)


def _proj_kernel(x_ref, w_ref, cosq_ref, sinq_ref, cosk_ref, sink_ref,
                 pool_ref, fc1w_ref, fc1b_ref, fc2w_ref, fc2b_ref,
                 qr_ref, ks_ref, vs_ref, kw_ref, vw_ref, kc_ref, vc_ref,
                 wg_ref):
    step = pl.program_id(0)
    x = x_ref[...]
    p = jax.lax.dot_general(x, w_ref[...], (((1,), (0,)), ((), ())),
                            preferred_element_type=jnp.float32)
    q = p[:, _OFF_Q:_OFF_KS]
    cq = cosq_ref[...]
    sq = sinq_ref[...]
    ck = cosk_ref[...]
    sk = sink_ref[...]
    qr_ref[...] = q * cq + _swap_halves(q, NH) * sq

    ks = p[:, _OFF_KS:_OFF_VS]
    ks_ref[...] = ks * ck + _swap_halves(ks, G) * sk
    vs_ref[...] = p[:, _OFF_VS:_OFF_KW]
    kw = p[:, _OFF_KW:_OFF_VW]
    kw_ref[...] = kw * ck + _swap_halves(kw, G) * sk
    vw_ref[...] = p[:, _OFF_VW:_OFF_KC]
    kcr = p[:, _OFF_KC:_OFF_VC]
    kcr = kcr * ck + _swap_halves(kcr, G) * sk
    pool = pool_ref[...]
    kc_part = jax.lax.dot_general(pool, kcr, (((1,), (0,)), ((), ())),
                                  preferred_element_type=jnp.float32,
                                  precision=jax.lax.Precision.HIGHEST)
    vc_part = jax.lax.dot_general(pool, p[:, _OFF_VC:_PTOT],
                                  (((1,), (0,)), ((), ())),
                                  preferred_element_type=jnp.float32,
                                  precision=jax.lax.Precision.HIGHEST)

    @pl.when(step == 0)
    def _init():
        kc_ref[...] = kc_part
        vc_ref[...] = vc_part

    @pl.when(step > 0)
    def _acc():
        kc_ref[...] += kc_part
        vc_ref[...] += vc_part

    @pl.when(step == pl.num_programs(0) - 1)
    def _finish():
        kc_ref[...] = kc_ref[...] * (1.0 / CL)
        vc_ref[...] = vc_ref[...] * (1.0 / CL)

    fc1w = fc1w_ref[...]
    fc1b = fc1b_ref[...]
    fc2w = fc2w_ref[...]
    fc2b = fc2b_ref[...]
    gate_cols = []
    for g in range(G):
        qg = q[:, g * HPG * DK:(g + 1) * HPG * DK]
        acc = qg[:, 0:DK]
        for h in range(1, HPG):
            acc = acc + qg[:, h * DK:(h + 1) * DK]
        qgp = acc / float(HPG)
        h1 = _bf16_dot(qgp, fc1w, (((1,), (1,)), ((), ()))) + fc1b
        h1 = h1 * jax.nn.sigmoid(h1)
        gl = _bf16_dot(h1, fc2w, (((1,), (1,)), ((), ()))) + fc2b
        x0 = gl[:, 0:1]
        x1 = gl[:, 1:2]
        x2 = gl[:, 2:3]
        mx = jnp.maximum(jnp.maximum(x0, x1), x2)
        mn = jnp.minimum(jnp.minimum(x0, x1), x2)
        mid = x0 + x1 + x2 - mx - mn
        e0 = jnp.exp(x0 - mx)
        e1 = jnp.exp(x1 - mx)
        e2 = jnp.exp(x2 - mx)
        z = e0 + e1 + e2
        peaked = (mx - mid) > 50.0
        a0 = x0 == mx
        a1 = (x1 == mx) & (~a0)
        a2 = (x2 == mx) & (~a0) & (~a1)
        w0 = jnp.where(peaked, a0.astype(jnp.float32), e0 / z)
        w1 = jnp.where(peaked, a1.astype(jnp.float32), e1 / z)
        w2 = jnp.where(peaked, a2.astype(jnp.float32), e2 / z)
        gate_cols += [w0, w1, w2]
    gate_cols.append(jnp.zeros((x.shape[0], 2), jnp.float32))
    wg_ref[...] = jnp.concatenate(gate_cols, axis=1)


def _masked_softmax(scores, mask):
    sm = jnp.where(mask, scores, NEG)
    mx = jnp.max(sm, axis=-1, keepdims=True)
    p = jnp.where(mask, jnp.exp(sm - mx), 0.0)
    denom = jnp.sum(p, axis=-1, keepdims=True)
    return jnp.where(denom > 0.0, p / jnp.where(denom > 0.0, denom, 1.0), 0.0)


def _softmax_ne(scores, mask):
    sm = jnp.where(mask, scores, NEG)
    mx = jnp.max(sm, axis=-1, keepdims=True)
    p = jnp.exp(sm - mx)
    return p / jnp.sum(p, axis=-1, keepdims=True)


def _attn_kernel(w, qoff, qr_ref, wg_ref, ks_ref, vs_ref, kw_ref, vw_ref,
                 kc_ref, vc_ref, ov_ref, exp_ref, wout_ref, out_ref):
    i = qoff + pl.program_id(0)
    q0 = i * QB
    t = q0 + jax.lax.broadcasted_iota(jnp.int32, (QB, 1), 0)
    ccol = jax.lax.broadcasted_iota(jnp.int32, (QB, CPAD), 1)
    bcol = jax.lax.broadcasted_iota(jnp.int32, (QB, NB), 1)

    m_cmp = ((ccol * CD + CL) <= (t + 1)) & (ccol < C)
    causal_blk = (bcol * LSEL) <= t
    forced = (bcol == 0) | (bcol == (t // LSEL))

    o_cmp_all = []
    sel_all = []
    gate_all = []
    for g in range(G):
        kcg = kc_ref[:, g * DK:(g + 1) * DK]
        vcg = vc_ref[:, g * DV:(g + 1) * DV]

        o_cmps = []
        psum = None
        for h in range(HPG):
            qh = qr_ref[:, (g * HPG + h) * DK:(g * HPG + h + 1) * DK]
            sc = _bf16_dot(qh, kcg, (((1,), (1,)), ((), ())))
            pc = _masked_softmax(sc * SCALE, m_cmp)
            psum = pc if psum is None else psum + pc
            o_cmps.append(
                _bf16_dot(pc, vcg, (((1,), (0,)), ((), ()))))
        o_cmp_all.append(o_cmps)

        p_slc = _bf16_dot(psum, ov_ref[...], (((1,), (0,)), ((), ())))
        score = jnp.where(causal_blk, p_slc, -1e9) + \
            jnp.where(forced, 1e6, 0.0)

        sel = jnp.zeros((QB, NB), jnp.float32)
        work = score
        for _ in range(NSEL):
            mx = jnp.max(work, axis=-1, keepdims=True)
            cand = jnp.where(work == mx, bcol, NB + 1)
            amin = jnp.min(cand, axis=-1, keepdims=True)
            pick = bcol == amin
            sel = jnp.where(pick, 1.0, sel)
            work = jnp.where(pick, -3e9, work)
        sel_all.append(sel)
        gate_all.append((wg_ref[:, g * 3:g * 3 + 1],
                         wg_ref[:, g * 3 + 1:g * 3 + 2],
                         wg_ref[:, g * 3 + 2:g * 3 + 3]))

    WWIN = WIN + QB
    kcol = jax.lax.broadcasted_iota(jnp.int32, (QB, w), 1)
    causal = kcol <= t
    wstart = jnp.maximum(i - 4, 0) * QB
    jw = wstart + jax.lax.broadcasted_iota(jnp.int32, (QB, WWIN), 1)
    win_m = (jw <= t) & (jw > t - WIN)
    o_parts = []
    for g in range(G):
        tok = _bf16_dot(sel_all[g], exp_ref[...], (((1,), (0,)), ((), ())))
        sel_mask = (tok > 0.5) & causal
        ksg = ks_ref[:, g * DK:(g + 1) * DK]
        vsg = vs_ref[:, g * DV:(g + 1) * DV]
        kwg = kw_ref[pl.ds(wstart, WWIN), g * DK:(g + 1) * DK]
        vwg = vw_ref[pl.ds(wstart, WWIN), g * DV:(g + 1) * DV]
        w_cmp, w_sel, w_win = gate_all[g]
        for h in range(HPG):
            hh = g * HPG + h
            qh = qr_ref[:, hh * DK:(hh + 1) * DK]
            ss = _bf16_dot(qh, ksg, (((1,), (1,)), ((), ())))
            ps = _softmax_ne(ss * SCALE, sel_mask)
            o_sel = _bf16_dot(ps, vsg, (((1,), (0,)), ((), ())))
            sw = _bf16_dot(qh, kwg, (((1,), (1,)), ((), ())))
            pw = _softmax_ne(sw * SCALE, win_m)
            o_win = _bf16_dot(pw, vwg, (((1,), (0,)), ((), ())))
            o_parts.append(w_cmp * o_cmp_all[g][h] + w_sel * o_sel
                           + w_win * o_win)

    o_all = jnp.concatenate(o_parts, axis=1)
    out_ref[...] = _bf16_dot(o_all, wout_ref[...], (((1,), (0,)), ((), ())))


def _rope_tables():
    pos = jnp.arange(S, dtype=jnp.float32)
    half = DK // 2
    freqs = 1.0 / (10000.0 ** (jnp.arange(half, dtype=jnp.float32) / half))
    ang = pos[:, None] * freqs[None, :]
    cos = jnp.cos(ang)
    sin = jnp.sin(ang)
    cos_h = jnp.concatenate([cos, cos], axis=1)
    sin_h = jnp.concatenate([-sin, sin], axis=1)
    cosq = jnp.tile(cos_h, (1, NH))
    sinq = jnp.tile(sin_h, (1, NH))
    cosk = jnp.tile(cos_h, (1, G))
    sink = jnp.tile(sin_h, (1, G))
    return cosq, sinq, cosk, sink


def _const_tables():
    pool = np.zeros((CPAD, S), dtype=np.float32)
    for c in range(C):
        pool[c, c * CD:c * CD + CL] = 1.0
    pool = jnp.asarray(pool)

    cstart = np.arange(C) * CD
    bstart = np.arange(NB) * LSEL
    ov = np.clip(np.minimum(cstart[:, None] + CL, bstart[None, :] + LSEL)
                 - np.maximum(cstart[:, None], bstart[None, :]),
                 0, None).astype(np.float32) / CL
    ovp = np.zeros((CPAD, NB), dtype=np.float32)
    ovp[:C] = ov
    ovp = jnp.asarray(ovp)

    expm = np.zeros((NB, S), dtype=np.float32)
    for bnum in range(NB):
        expm[bnum, bnum * LSEL:(bnum + 1) * LSEL] = 1.0
    expm = jnp.asarray(expm)
    return pool, ovp, expm


def _run_proj(x2, w_all, fc1_w, fc1_b, fc2_w, fc2_b):
    cosq, sinq, cosk, sink = _rope_tables()
    pool, _, _ = _const_tables()
    fc2w_pad = jnp.zeros((8, HID), jnp.float32).at[:3].set(fc2_w)
    fc2b_pad = jnp.zeros((1, 8), jnp.float32).at[0, :3].set(fc2_b)

    return pl.pallas_call(
        _proj_kernel,
        grid=(S // RB,),
        in_specs=[
            pl.BlockSpec((RB, DIM), lambda i: (i, 0)),
            pl.BlockSpec((DIM, _PTOT), lambda i: (0, 0)),
            pl.BlockSpec((RB, NH * DK), lambda i: (i, 0)),
            pl.BlockSpec((RB, NH * DK), lambda i: (i, 0)),
            pl.BlockSpec((RB, G * DK), lambda i: (i, 0)),
            pl.BlockSpec((RB, G * DK), lambda i: (i, 0)),
            pl.BlockSpec((CPAD, RB), lambda i: (0, i)),
            pl.BlockSpec((HID, DK), lambda i: (0, 0)),
            pl.BlockSpec((1, HID), lambda i: (0, 0)),
            pl.BlockSpec((8, HID), lambda i: (0, 0)),
            pl.BlockSpec((1, 8), lambda i: (0, 0)),
        ],
        out_specs=[
            pl.BlockSpec((RB, NH * DK), lambda i: (i, 0)),
            pl.BlockSpec((RB, G * DK), lambda i: (i, 0)),
            pl.BlockSpec((RB, G * DV), lambda i: (i, 0)),
            pl.BlockSpec((RB, G * DK), lambda i: (i, 0)),
            pl.BlockSpec((RB, G * DV), lambda i: (i, 0)),
            pl.BlockSpec((CPAD, G * DK), lambda i: (0, 0)),
            pl.BlockSpec((CPAD, G * DV), lambda i: (0, 0)),
            pl.BlockSpec((RB, 8), lambda i: (i, 0)),
        ],
        out_shape=[
            jax.ShapeDtypeStruct((S, NH * DK), jnp.float32),
            jax.ShapeDtypeStruct((S, G * DK), jnp.float32),
            jax.ShapeDtypeStruct((S, G * DV), jnp.float32),
            jax.ShapeDtypeStruct((S, G * DK), jnp.float32),
            jax.ShapeDtypeStruct((S, G * DV), jnp.float32),
            jax.ShapeDtypeStruct((CPAD, G * DK), jnp.float32),
            jax.ShapeDtypeStruct((CPAD, G * DV), jnp.float32),
            jax.ShapeDtypeStruct((S, 8), jnp.float32),
        ],
    )(x2.astype(jnp.bfloat16), w_all.astype(jnp.bfloat16),
      cosq, sinq, cosk, sink, pool,
      fc1_w.astype(jnp.bfloat16), fc1_b.reshape(1, HID),
      fc2w_pad.astype(jnp.bfloat16), fc2b_pad)


def _run_attn(qr, wg, ks, vs, kw, vw, kc, vc, W_out):
    _, ovp, expm = _const_tables()
    qr = qr.astype(jnp.bfloat16)
    ks = ks.astype(jnp.bfloat16)
    vs = vs.astype(jnp.bfloat16)
    kw = kw.astype(jnp.bfloat16)
    vw = vw.astype(jnp.bfloat16)
    kc = kc.astype(jnp.bfloat16)
    vc = vc.astype(jnp.bfloat16)
    ovp = ovp.astype(jnp.bfloat16)
    expm = expm.astype(jnp.bfloat16)
    wout = W_out.T.astype(jnp.bfloat16)

    ncalls = 4
    per = S // QB // ncalls
    parts = []
    for c in range(ncalls):
        w = (c + 1) * per * QB
        parts.append(pl.pallas_call(
            functools.partial(_attn_kernel, w, c * per),
            grid=(per,),
            in_specs=[
                pl.BlockSpec((QB, NH * DK), lambda i, c=c: (c * per + i, 0)),
                pl.BlockSpec((QB, 8), lambda i, c=c: (c * per + i, 0)),
                pl.BlockSpec((w, G * DK), lambda i: (0, 0)),
                pl.BlockSpec((w, G * DV), lambda i: (0, 0)),
                pl.BlockSpec((S, G * DK), lambda i: (0, 0)),
                pl.BlockSpec((S, G * DV), lambda i: (0, 0)),
                pl.BlockSpec((CPAD, G * DK), lambda i: (0, 0)),
                pl.BlockSpec((CPAD, G * DV), lambda i: (0, 0)),
                pl.BlockSpec((CPAD, NB), lambda i: (0, 0)),
                pl.BlockSpec((NB, w), lambda i: (0, 0)),
                pl.BlockSpec((NH * DV, DIM), lambda i: (0, 0)),
            ],
            out_specs=pl.BlockSpec((QB, DIM), lambda i: (i, 0)),
            out_shape=jax.ShapeDtypeStruct((per * QB, DIM), jnp.float32),
        )(qr, wg, ks[:w], vs[:w], kw, vw, kc, vc, ovp, expm[:, :w], wout))
    return jnp.concatenate(parts, axis=0)


@functools.partial(jax.jit, static_argnames=())
def kernel(x, W_Q, W_K_sel, W_V_sel, W_K_win, W_V_win, W_K_cmp, W_V_cmp,
           W_out, fc1_w, fc1_b, fc2_w, fc2_b):
    x2 = x.reshape(S, DIM)
    w_all = jnp.concatenate(
        [W_Q, W_K_sel, W_V_sel, W_K_win, W_V_win, W_K_cmp, W_V_cmp],
        axis=0).T
    qr, ks, vs, kw, vw, kc, vc, wg = _run_proj(
        x2, w_all, fc1_w, fc1_b, fc2_w, fc2_b)
    out = _run_attn(qr, wg, ks, vs, kw, vw, kc, vc, W_out)
    return out.reshape(B, S, DIM)


# speedup vs baseline: 2.3674x; 1.2637x over previous
import functools

import jax
import jax.numpy as jnp
import numpy as np
from jax.experimental import pallas as pl
from jax.experimental.pallas import tpu as pltpu

B = 1
S = 2048
DIM = 768
NH = 12
G = 2
HPG = NH // G
DK = 64
DV = 64
CL = 32
CD = 16
LSEL = 64
NSEL = 8
WIN = 512
TAU = 1.0
HID = DK // 2
C = (S - CL) // CD + 1
CPAD = 128
NB = S // LSEL
QB = 256
NEG = -1e30
SCALE = 1.0 / 8.0

_OFF_Q = 0
_OFF_KS = NH * DK
_OFF_VS = _OFF_KS + G * DK
_OFF_KW = _OFF_VS + G * DV
_OFF_VW = _OFF_KW + G * DK
_OFF_KC = _OFF_VW + G * DV
_OFF_VC = _OFF_KC + G * DK
_PTOT = _OFF_VC + G * DV


def _swap_halves(xx, nheads):
    parts = []
    for h in range(nheads):
        base = h * DK
        parts.append(xx[:, base + DK // 2: base + DK])
        parts.append(xx[:, base: base + DK // 2])
    return jnp.concatenate(parts, axis=1)


RB = 512


def _bf16_dot(a, b, dn):
    return jax.lax.dot_general(a.astype(jnp.bfloat16), b.astype(jnp.bfloat16),
                               dn, preferred_element_type=jnp.float32)


def _proj_kernel(x_ref, w_ref, cosq_ref, sinq_ref, cosk_ref, sink_ref,
                 pool_ref, fc1w_ref, fc1b_ref, fc2w_ref, fc2b_ref,
                 qr_ref, ks_ref, vs_ref, kw_ref, vw_ref, kc_ref, vc_ref,
                 wg_ref):
    step = pl.program_id(0)
    x = x_ref[...]
    p = jax.lax.dot_general(x, w_ref[...], (((1,), (0,)), ((), ())),
                            preferred_element_type=jnp.float32)
    q = p[:, _OFF_Q:_OFF_KS]
    cq = cosq_ref[...]
    sq = sinq_ref[...]
    ck = cosk_ref[...]
    sk = sink_ref[...]
    qr_ref[...] = (q * cq + _swap_halves(q, NH) * sq).astype(jnp.bfloat16)

    ks = p[:, _OFF_KS:_OFF_VS]
    ks_ref[...] = (ks * ck + _swap_halves(ks, G) * sk).astype(jnp.bfloat16)
    vs_ref[...] = p[:, _OFF_VS:_OFF_KW].astype(jnp.bfloat16)
    kw = p[:, _OFF_KW:_OFF_VW]
    kw_ref[...] = (kw * ck + _swap_halves(kw, G) * sk).astype(jnp.bfloat16)
    vw_ref[...] = p[:, _OFF_VW:_OFF_KC].astype(jnp.bfloat16)
    kcr = p[:, _OFF_KC:_OFF_VC]
    kcr = kcr * ck + _swap_halves(kcr, G) * sk
    pool = pool_ref[...]
    kc_part = jax.lax.dot_general(pool, kcr, (((1,), (0,)), ((), ())),
                                  preferred_element_type=jnp.float32,
                                  precision=jax.lax.Precision.HIGHEST)
    vc_part = jax.lax.dot_general(pool, p[:, _OFF_VC:_PTOT],
                                  (((1,), (0,)), ((), ())),
                                  preferred_element_type=jnp.float32,
                                  precision=jax.lax.Precision.HIGHEST)

    @pl.when(step == 0)
    def _init():
        kc_ref[...] = kc_part
        vc_ref[...] = vc_part

    @pl.when(step > 0)
    def _acc():
        kc_ref[...] += kc_part
        vc_ref[...] += vc_part

    @pl.when(step == pl.num_programs(0) - 1)
    def _finish():
        kc_ref[...] = kc_ref[...] * (1.0 / CL)
        vc_ref[...] = vc_ref[...] * (1.0 / CL)

    fc1w = fc1w_ref[...]
    fc1b = fc1b_ref[...]
    fc2w = fc2w_ref[...]
    fc2b = fc2b_ref[...]
    gate_cols = []
    for g in range(G):
        qg = q[:, g * HPG * DK:(g + 1) * HPG * DK]
        acc = qg[:, 0:DK]
        for h in range(1, HPG):
            acc = acc + qg[:, h * DK:(h + 1) * DK]
        qgp = acc / float(HPG)
        h1 = _bf16_dot(qgp, fc1w, (((1,), (1,)), ((), ()))) + fc1b
        h1 = h1 * jax.nn.sigmoid(h1)
        gl = _bf16_dot(h1, fc2w, (((1,), (1,)), ((), ()))) + fc2b
        x0 = gl[:, 0:1]
        x1 = gl[:, 1:2]
        x2 = gl[:, 2:3]
        mx = jnp.maximum(jnp.maximum(x0, x1), x2)
        mn = jnp.minimum(jnp.minimum(x0, x1), x2)
        mid = x0 + x1 + x2 - mx - mn
        e0 = jnp.exp(x0 - mx)
        e1 = jnp.exp(x1 - mx)
        e2 = jnp.exp(x2 - mx)
        z = e0 + e1 + e2
        peaked = (mx - mid) > 50.0
        a0 = x0 == mx
        a1 = (x1 == mx) & (~a0)
        a2 = (x2 == mx) & (~a0) & (~a1)
        w0 = jnp.where(peaked, a0.astype(jnp.float32), e0 / z)
        w1 = jnp.where(peaked, a1.astype(jnp.float32), e1 / z)
        w2 = jnp.where(peaked, a2.astype(jnp.float32), e2 / z)
        gate_cols += [w0, w1, w2]
    gate_cols.append(jnp.zeros((x.shape[0], 2), jnp.float32))
    wg_ref[...] = jnp.concatenate(gate_cols, axis=1)


def _masked_softmax(scores, mask):
    sm = jnp.where(mask, scores, NEG)
    mx = jnp.max(sm, axis=-1, keepdims=True)
    p = jnp.where(mask, jnp.exp(sm - mx), 0.0)
    denom = jnp.sum(p, axis=-1, keepdims=True)
    return jnp.where(denom > 0.0, p / jnp.where(denom > 0.0, denom, 1.0), 0.0)


def _softmax_ne(scores, mask):
    sm = jnp.where(mask, scores, NEG)
    mx = jnp.max(sm, axis=-1, keepdims=True)
    p = jnp.exp(sm - mx)
    return p / jnp.sum(p, axis=-1, keepdims=True)


def _attn_kernel(w, qoff, qr_ref, wg_ref, ks_ref, vs_ref, kw_ref, vw_ref,
                 kc_ref, vc_ref, ov_ref, exp_ref, wout_ref, out_ref):
    i = qoff + pl.program_id(0)
    q0 = i * QB
    t = q0 + jax.lax.broadcasted_iota(jnp.int32, (QB, 1), 0)
    ccol = jax.lax.broadcasted_iota(jnp.int32, (QB, CPAD), 1)
    bcol = jax.lax.broadcasted_iota(jnp.int32, (QB, NB), 1)

    m_cmp = ((ccol * CD + CL) <= (t + 1)) & (ccol < C)
    causal_blk = (bcol * LSEL) <= t
    forced = (bcol == 0) | (bcol == (t // LSEL))

    o_cmp_all = []
    sel_all = []
    gate_all = []
    for g in range(G):
        kcg = kc_ref[:, g * DK:(g + 1) * DK]
        vcg = vc_ref[:, g * DV:(g + 1) * DV]

        o_cmps = []
        psum = None
        for h in range(HPG):
            qh = qr_ref[:, (g * HPG + h) * DK:(g * HPG + h + 1) * DK]
            sc = _bf16_dot(qh, kcg, (((1,), (1,)), ((), ())))
            pc = _masked_softmax(sc * SCALE, m_cmp)
            psum = pc if psum is None else psum + pc
            o_cmps.append(
                _bf16_dot(pc, vcg, (((1,), (0,)), ((), ()))))
        o_cmp_all.append(o_cmps)

        p_slc = _bf16_dot(psum, ov_ref[...], (((1,), (0,)), ((), ())))
        score = jnp.where(causal_blk, p_slc, -1e9) + \
            jnp.where(forced, 1e6, 0.0)

        sel = jnp.zeros((QB, NB), jnp.float32)
        work = score
        for _ in range(NSEL):
            mx = jnp.max(work, axis=-1, keepdims=True)
            cand = jnp.where(work == mx, bcol, NB + 1)
            amin = jnp.min(cand, axis=-1, keepdims=True)
            pick = bcol == amin
            sel = jnp.where(pick, 1.0, sel)
            work = jnp.where(pick, -3e9, work)
        sel_all.append(sel)
        gate_all.append((wg_ref[:, g * 3:g * 3 + 1],
                         wg_ref[:, g * 3 + 1:g * 3 + 2],
                         wg_ref[:, g * 3 + 2:g * 3 + 3]))

    WWIN = WIN + QB
    kcol = jax.lax.broadcasted_iota(jnp.int32, (QB, w), 1)
    causal = kcol <= t
    wstart = jnp.maximum(i - WIN // QB, 0) * QB
    jw = wstart + jax.lax.broadcasted_iota(jnp.int32, (QB, WWIN), 1)
    win_m = (jw <= t) & (jw > t - WIN)
    o_parts = []
    for g in range(G):
        tok = _bf16_dot(sel_all[g], exp_ref[...], (((1,), (0,)), ((), ())))
        sel_mask = (tok > 0.5) & causal
        ksg = ks_ref[:, g * DK:(g + 1) * DK]
        vsg = vs_ref[:, g * DV:(g + 1) * DV]
        kwg = kw_ref[pl.ds(wstart, WWIN), g * DK:(g + 1) * DK]
        vwg = vw_ref[pl.ds(wstart, WWIN), g * DV:(g + 1) * DV]
        w_cmp, w_sel, w_win = gate_all[g]
        for h in range(HPG):
            hh = g * HPG + h
            qh = qr_ref[:, hh * DK:(hh + 1) * DK]
            ss = _bf16_dot(qh, ksg, (((1,), (1,)), ((), ())))
            ps = _softmax_ne(ss * SCALE, sel_mask)
            o_sel = _bf16_dot(ps, vsg, (((1,), (0,)), ((), ())))
            sw = _bf16_dot(qh, kwg, (((1,), (1,)), ((), ())))
            pw = _softmax_ne(sw * SCALE, win_m)
            o_win = _bf16_dot(pw, vwg, (((1,), (0,)), ((), ())))
            o_parts.append(w_cmp * o_cmp_all[g][h] + w_sel * o_sel
                           + w_win * o_win)

    o_all = jnp.concatenate(o_parts, axis=1)
    out_ref[...] = _bf16_dot(o_all, wout_ref[...], (((1,), (0,)), ((), ())))


def _rope_tables():
    pos = jnp.arange(S, dtype=jnp.float32)
    half = DK // 2
    freqs = 1.0 / (10000.0 ** (jnp.arange(half, dtype=jnp.float32) / half))
    ang = pos[:, None] * freqs[None, :]
    cos = jnp.cos(ang)
    sin = jnp.sin(ang)
    cos_h = jnp.concatenate([cos, cos], axis=1)
    sin_h = jnp.concatenate([-sin, sin], axis=1)
    cosq = jnp.tile(cos_h, (1, NH))
    sinq = jnp.tile(sin_h, (1, NH))
    cosk = jnp.tile(cos_h, (1, G))
    sink = jnp.tile(sin_h, (1, G))
    return cosq, sinq, cosk, sink


def _const_tables():
    pool = np.zeros((CPAD, S), dtype=np.float32)
    for c in range(C):
        pool[c, c * CD:c * CD + CL] = 1.0
    pool = jnp.asarray(pool)

    cstart = np.arange(C) * CD
    bstart = np.arange(NB) * LSEL
    ov = np.clip(np.minimum(cstart[:, None] + CL, bstart[None, :] + LSEL)
                 - np.maximum(cstart[:, None], bstart[None, :]),
                 0, None).astype(np.float32) / CL
    ovp = np.zeros((CPAD, NB), dtype=np.float32)
    ovp[:C] = ov
    ovp = jnp.asarray(ovp)

    expm = np.zeros((NB, S), dtype=np.float32)
    for bnum in range(NB):
        expm[bnum, bnum * LSEL:(bnum + 1) * LSEL] = 1.0
    expm = jnp.asarray(expm)
    return pool, ovp, expm


def _run_proj(x2, w_all, fc1_w, fc1_b, fc2_w, fc2_b):
    cosq, sinq, cosk, sink = _rope_tables()
    pool, _, _ = _const_tables()
    fc2w_pad = jnp.zeros((8, HID), jnp.float32).at[:3].set(fc2_w)
    fc2b_pad = jnp.zeros((1, 8), jnp.float32).at[0, :3].set(fc2_b)

    return pl.pallas_call(
        _proj_kernel,
        grid=(S // RB,),
        in_specs=[
            pl.BlockSpec((RB, DIM), lambda i: (i, 0)),
            pl.BlockSpec((DIM, _PTOT), lambda i: (0, 0)),
            pl.BlockSpec((RB, NH * DK), lambda i: (i, 0)),
            pl.BlockSpec((RB, NH * DK), lambda i: (i, 0)),
            pl.BlockSpec((RB, G * DK), lambda i: (i, 0)),
            pl.BlockSpec((RB, G * DK), lambda i: (i, 0)),
            pl.BlockSpec((CPAD, RB), lambda i: (0, i)),
            pl.BlockSpec((HID, DK), lambda i: (0, 0)),
            pl.BlockSpec((1, HID), lambda i: (0, 0)),
            pl.BlockSpec((8, HID), lambda i: (0, 0)),
            pl.BlockSpec((1, 8), lambda i: (0, 0)),
        ],
        out_specs=[
            pl.BlockSpec((RB, NH * DK), lambda i: (i, 0)),
            pl.BlockSpec((RB, G * DK), lambda i: (i, 0)),
            pl.BlockSpec((RB, G * DV), lambda i: (i, 0)),
            pl.BlockSpec((RB, G * DK), lambda i: (i, 0)),
            pl.BlockSpec((RB, G * DV), lambda i: (i, 0)),
            pl.BlockSpec((CPAD, G * DK), lambda i: (0, 0)),
            pl.BlockSpec((CPAD, G * DV), lambda i: (0, 0)),
            pl.BlockSpec((RB, 8), lambda i: (i, 0)),
        ],
        out_shape=[
            jax.ShapeDtypeStruct((S, NH * DK), jnp.bfloat16),
            jax.ShapeDtypeStruct((S, G * DK), jnp.bfloat16),
            jax.ShapeDtypeStruct((S, G * DV), jnp.bfloat16),
            jax.ShapeDtypeStruct((S, G * DK), jnp.bfloat16),
            jax.ShapeDtypeStruct((S, G * DV), jnp.bfloat16),
            jax.ShapeDtypeStruct((CPAD, G * DK), jnp.float32),
            jax.ShapeDtypeStruct((CPAD, G * DV), jnp.float32),
            jax.ShapeDtypeStruct((S, 8), jnp.float32),
        ],
    )(x2.astype(jnp.bfloat16), w_all.astype(jnp.bfloat16),
      cosq, sinq, cosk, sink, pool,
      fc1_w.astype(jnp.bfloat16), fc1_b.reshape(1, HID),
      fc2w_pad.astype(jnp.bfloat16), fc2b_pad)


def _run_attn(qr, wg, ks, vs, kw, vw, kc, vc, W_out):
    _, ovp, expm = _const_tables()
    qr = qr.astype(jnp.bfloat16)
    ks = ks.astype(jnp.bfloat16)
    vs = vs.astype(jnp.bfloat16)
    kw = kw.astype(jnp.bfloat16)
    vw = vw.astype(jnp.bfloat16)
    kc = kc.astype(jnp.bfloat16)
    vc = vc.astype(jnp.bfloat16)
    ovp = ovp.astype(jnp.bfloat16)
    expm = expm.astype(jnp.bfloat16)
    wout = W_out.T.astype(jnp.bfloat16)

    ncalls = 4
    per = S // QB // ncalls
    parts = []
    for c in range(ncalls):
        w = (c + 1) * per * QB
        parts.append(pl.pallas_call(
            functools.partial(_attn_kernel, w, c * per),
            grid=(per,),
            in_specs=[
                pl.BlockSpec((QB, NH * DK), lambda i, c=c: (c * per + i, 0)),
                pl.BlockSpec((QB, 8), lambda i, c=c: (c * per + i, 0)),
                pl.BlockSpec((w, G * DK), lambda i: (0, 0)),
                pl.BlockSpec((w, G * DV), lambda i: (0, 0)),
                pl.BlockSpec((S, G * DK), lambda i: (0, 0)),
                pl.BlockSpec((S, G * DV), lambda i: (0, 0)),
                pl.BlockSpec((CPAD, G * DK), lambda i: (0, 0)),
                pl.BlockSpec((CPAD, G * DV), lambda i: (0, 0)),
                pl.BlockSpec((CPAD, NB), lambda i: (0, 0)),
                pl.BlockSpec((NB, w), lambda i: (0, 0)),
                pl.BlockSpec((NH * DV, DIM), lambda i: (0, 0)),
            ],
            out_specs=pl.BlockSpec((QB, DIM), lambda i: (i, 0)),
            out_shape=jax.ShapeDtypeStruct((per * QB, DIM), jnp.float32),
        )(qr, wg, ks[:w], vs[:w], kw, vw, kc, vc, ovp, expm[:, :w], wout))
    return jnp.concatenate(parts, axis=0)


@functools.partial(jax.jit, static_argnames=())
def kernel(x, W_Q, W_K_sel, W_V_sel, W_K_win, W_V_win, W_K_cmp, W_V_cmp,
           W_out, fc1_w, fc1_b, fc2_w, fc2_b):
    x2 = x.reshape(S, DIM)
    w_all = jnp.concatenate(
        [W_Q, W_K_sel, W_V_sel, W_K_win, W_V_win, W_K_cmp, W_V_cmp],
        axis=0).T
    qr, ks, vs, kw, vw, kc, vc, wg = _run_proj(
        x2, w_all, fc1_w, fc1_b, fc2_w, fc2_b)
    out = _run_attn(qr, wg, ks, vs, kw, vw, kc, vc, W_out)
    return out.reshape(B, S, DIM)


# speedup vs baseline: 3.2373x; 1.3674x over previous
import functools

import jax
import jax.numpy as jnp
import numpy as np
from jax.experimental import pallas as pl
from jax.experimental.pallas import tpu as pltpu

B = 1
S = 2048
DIM = 768
NH = 12
G = 2
HPG = NH // G
DK = 64
DV = 64
CL = 32
CD = 16
LSEL = 64
NSEL = 8
WIN = 512
TAU = 1.0
HID = DK // 2
C = (S - CL) // CD + 1
CPAD = 128
NB = S // LSEL
QB = 512
NEG = -1e30
SCALE = 1.0 / 8.0

_OFF_Q = 0
_OFF_KS = NH * DK
_OFF_VS = _OFF_KS + G * DK
_OFF_KW = _OFF_VS + G * DV
_OFF_VW = _OFF_KW + G * DK
_OFF_KC = _OFF_VW + G * DV
_OFF_VC = _OFF_KC + G * DK
_PTOT = _OFF_VC + G * DV


def _swap_halves(xx, nheads):
    parts = []
    for h in range(nheads):
        base = h * DK
        parts.append(xx[:, base + DK // 2: base + DK])
        parts.append(xx[:, base: base + DK // 2])
    return jnp.concatenate(parts, axis=1)


RB = 512


def _bf16_dot(a, b, dn):
    return jax.lax.dot_general(a.astype(jnp.bfloat16), b.astype(jnp.bfloat16),
                               dn, preferred_element_type=jnp.float32)


def _proj_kernel(x_ref, w_ref, cosq_ref, sinq_ref, cosk_ref, sink_ref,
                 pool_ref, fc1w_ref, fc1b_ref, fc2w_ref, fc2b_ref,
                 qr_ref, ks_ref, vs_ref, kw_ref, vw_ref, kc_ref, vc_ref,
                 wg_ref):
    step = pl.program_id(0)
    x = x_ref[...]
    p = jax.lax.dot_general(x, w_ref[...], (((1,), (0,)), ((), ())),
                            preferred_element_type=jnp.float32)
    q = p[:, _OFF_Q:_OFF_KS]
    cq = cosq_ref[...]
    sq = sinq_ref[...]
    ck = cosk_ref[...]
    sk = sink_ref[...]
    qr_ref[...] = (q * cq + _swap_halves(q, NH) * sq).astype(jnp.bfloat16)

    ks = p[:, _OFF_KS:_OFF_VS]
    ks_ref[...] = (ks * ck + _swap_halves(ks, G) * sk).astype(jnp.bfloat16)
    vs_ref[...] = p[:, _OFF_VS:_OFF_KW].astype(jnp.bfloat16)
    kw = p[:, _OFF_KW:_OFF_VW]
    kw_ref[...] = (kw * ck + _swap_halves(kw, G) * sk).astype(jnp.bfloat16)
    vw_ref[...] = p[:, _OFF_VW:_OFF_KC].astype(jnp.bfloat16)
    kcr = p[:, _OFF_KC:_OFF_VC]
    kcr = kcr * ck + _swap_halves(kcr, G) * sk
    pool = pool_ref[...]
    kc_part = jax.lax.dot_general(pool, kcr, (((1,), (0,)), ((), ())),
                                  preferred_element_type=jnp.float32,
                                  precision=jax.lax.Precision.HIGHEST)
    vc_part = jax.lax.dot_general(pool, p[:, _OFF_VC:_PTOT],
                                  (((1,), (0,)), ((), ())),
                                  preferred_element_type=jnp.float32,
                                  precision=jax.lax.Precision.HIGHEST)

    @pl.when(step == 0)
    def _init():
        kc_ref[...] = kc_part
        vc_ref[...] = vc_part

    @pl.when(step > 0)
    def _acc():
        kc_ref[...] += kc_part
        vc_ref[...] += vc_part

    @pl.when(step == pl.num_programs(0) - 1)
    def _finish():
        kc_ref[...] = kc_ref[...] * (1.0 / CL)
        vc_ref[...] = vc_ref[...] * (1.0 / CL)

    fc1w = fc1w_ref[...]
    fc1b = fc1b_ref[...]
    fc2w = fc2w_ref[...]
    fc2b = fc2b_ref[...]
    gate_cols = []
    for g in range(G):
        qg = q[:, g * HPG * DK:(g + 1) * HPG * DK]
        acc = qg[:, 0:DK]
        for h in range(1, HPG):
            acc = acc + qg[:, h * DK:(h + 1) * DK]
        qgp = acc / float(HPG)
        h1 = _bf16_dot(qgp, fc1w, (((1,), (1,)), ((), ()))) + fc1b
        h1 = h1 * jax.nn.sigmoid(h1)
        gl = _bf16_dot(h1, fc2w, (((1,), (1,)), ((), ()))) + fc2b
        x0 = gl[:, 0:1]
        x1 = gl[:, 1:2]
        x2 = gl[:, 2:3]
        mx = jnp.maximum(jnp.maximum(x0, x1), x2)
        mn = jnp.minimum(jnp.minimum(x0, x1), x2)
        mid = x0 + x1 + x2 - mx - mn
        e0 = jnp.exp(x0 - mx)
        e1 = jnp.exp(x1 - mx)
        e2 = jnp.exp(x2 - mx)
        z = e0 + e1 + e2
        peaked = (mx - mid) > 50.0
        a0 = x0 == mx
        a1 = (x1 == mx) & (~a0)
        a2 = (x2 == mx) & (~a0) & (~a1)
        w0 = jnp.where(peaked, a0.astype(jnp.float32), e0 / z)
        w1 = jnp.where(peaked, a1.astype(jnp.float32), e1 / z)
        w2 = jnp.where(peaked, a2.astype(jnp.float32), e2 / z)
        gate_cols += [w0, w1, w2]
    gate_cols.append(jnp.zeros((x.shape[0], 2), jnp.float32))
    wg_ref[...] = jnp.concatenate(gate_cols, axis=1)


def _masked_softmax(scores, mask):
    sm = jnp.where(mask, scores, NEG)
    mx = jnp.max(sm, axis=-1, keepdims=True)
    p = jnp.where(mask, jnp.exp(sm - mx), 0.0)
    denom = jnp.sum(p, axis=-1, keepdims=True)
    return jnp.where(denom > 0.0, p / jnp.where(denom > 0.0, denom, 1.0), 0.0)


def _softmax_ne(scores, mask):
    sm = jnp.where(mask, scores, NEG)
    mx = jnp.max(sm, axis=-1, keepdims=True)
    p = jnp.exp(sm - mx)
    return p / jnp.sum(p, axis=-1, keepdims=True)


def _attn_kernel(w, qoff, qr_ref, wg_ref, ks_ref, vs_ref, kw_ref, vw_ref,
                 kc_ref, vc_ref, ov_ref, exp_ref, wout_ref, out_ref):
    i = qoff + pl.program_id(0)
    q0 = i * QB
    t = q0 + jax.lax.broadcasted_iota(jnp.int32, (QB, 1), 0)
    ccol = jax.lax.broadcasted_iota(jnp.int32, (QB, CPAD), 1)
    bcol = jax.lax.broadcasted_iota(jnp.int32, (QB, NB), 1)

    m_cmp = ((ccol * CD + CL) <= (t + 1)) & (ccol < C)
    causal_blk = (bcol * LSEL) <= t
    forced = (bcol == 0) | (bcol == (t // LSEL))

    o_cmp_all = []
    sel_all = []
    gate_all = []
    for g in range(G):
        kcg = kc_ref[:, g * DK:(g + 1) * DK]
        vcg = vc_ref[:, g * DV:(g + 1) * DV]

        o_cmps = []
        psum = None
        for h in range(HPG):
            qh = qr_ref[:, (g * HPG + h) * DK:(g * HPG + h + 1) * DK]
            sc = _bf16_dot(qh, kcg, (((1,), (1,)), ((), ())))
            pc = _masked_softmax(sc * SCALE, m_cmp)
            psum = pc if psum is None else psum + pc
            o_cmps.append(
                _bf16_dot(pc, vcg, (((1,), (0,)), ((), ()))))
        o_cmp_all.append(o_cmps)

        p_slc = _bf16_dot(psum, ov_ref[...], (((1,), (0,)), ((), ())))
        score = jnp.where(causal_blk, p_slc, -1e9) + \
            jnp.where(forced, 1e6, 0.0)

        sel = jnp.zeros((QB, NB), jnp.float32)
        work = score
        for _ in range(NSEL):
            mx = jnp.max(work, axis=-1, keepdims=True)
            cand = jnp.where(work == mx, bcol, NB + 1)
            amin = jnp.min(cand, axis=-1, keepdims=True)
            pick = bcol == amin
            sel = jnp.where(pick, 1.0, sel)
            work = jnp.where(pick, -3e9, work)
        sel_all.append(sel)
        gate_all.append((wg_ref[:, g * 3:g * 3 + 1],
                         wg_ref[:, g * 3 + 1:g * 3 + 2],
                         wg_ref[:, g * 3 + 2:g * 3 + 3]))

    WWIN = WIN + QB
    kcol = jax.lax.broadcasted_iota(jnp.int32, (QB, w), 1)
    causal = kcol <= t
    wstart = jnp.maximum(i - WIN // QB, 0) * QB
    jw = wstart + jax.lax.broadcasted_iota(jnp.int32, (QB, WWIN), 1)
    win_m = (jw <= t) & (jw > t - WIN)
    o_parts = []
    for g in range(G):
        tok = _bf16_dot(sel_all[g], exp_ref[...], (((1,), (0,)), ((), ())))
        sel_mask = (tok > 0.5) & causal
        ksg = ks_ref[:, g * DK:(g + 1) * DK]
        vsg = vs_ref[:, g * DV:(g + 1) * DV]
        kwg = kw_ref[pl.ds(wstart, WWIN), g * DK:(g + 1) * DK]
        vwg = vw_ref[pl.ds(wstart, WWIN), g * DV:(g + 1) * DV]
        w_cmp, w_sel, w_win = gate_all[g]
        for h in range(HPG):
            hh = g * HPG + h
            qh = qr_ref[:, hh * DK:(hh + 1) * DK]
            ss = _bf16_dot(qh, ksg, (((1,), (1,)), ((), ())))
            ps = _softmax_ne(ss * SCALE, sel_mask)
            o_sel = _bf16_dot(ps, vsg, (((1,), (0,)), ((), ())))
            sw = _bf16_dot(qh, kwg, (((1,), (1,)), ((), ())))
            pw = _softmax_ne(sw * SCALE, win_m)
            o_win = _bf16_dot(pw, vwg, (((1,), (0,)), ((), ())))
            o_parts.append(w_cmp * o_cmp_all[g][h] + w_sel * o_sel
                           + w_win * o_win)

    o_all = jnp.concatenate(o_parts, axis=1)
    out_ref[...] = _bf16_dot(o_all, wout_ref[...], (((1,), (0,)), ((), ())))


def _rope_tables():
    pos = jnp.arange(S, dtype=jnp.float32)
    half = DK // 2
    freqs = 1.0 / (10000.0 ** (jnp.arange(half, dtype=jnp.float32) / half))
    ang = pos[:, None] * freqs[None, :]
    cos = jnp.cos(ang)
    sin = jnp.sin(ang)
    cos_h = jnp.concatenate([cos, cos], axis=1)
    sin_h = jnp.concatenate([-sin, sin], axis=1)
    cosq = jnp.tile(cos_h, (1, NH))
    sinq = jnp.tile(sin_h, (1, NH))
    cosk = jnp.tile(cos_h, (1, G))
    sink = jnp.tile(sin_h, (1, G))
    return cosq, sinq, cosk, sink


def _const_tables():
    pool = np.zeros((CPAD, S), dtype=np.float32)
    for c in range(C):
        pool[c, c * CD:c * CD + CL] = 1.0
    pool = jnp.asarray(pool)

    cstart = np.arange(C) * CD
    bstart = np.arange(NB) * LSEL
    ov = np.clip(np.minimum(cstart[:, None] + CL, bstart[None, :] + LSEL)
                 - np.maximum(cstart[:, None], bstart[None, :]),
                 0, None).astype(np.float32) / CL
    ovp = np.zeros((CPAD, NB), dtype=np.float32)
    ovp[:C] = ov
    ovp = jnp.asarray(ovp)

    expm = np.zeros((NB, S), dtype=np.float32)
    for bnum in range(NB):
        expm[bnum, bnum * LSEL:(bnum + 1) * LSEL] = 1.0
    expm = jnp.asarray(expm)
    return pool, ovp, expm


def _run_proj(x2, w_all, fc1_w, fc1_b, fc2_w, fc2_b):
    cosq, sinq, cosk, sink = _rope_tables()
    pool, _, _ = _const_tables()
    fc2w_pad = jnp.zeros((8, HID), jnp.float32).at[:3].set(fc2_w)
    fc2b_pad = jnp.zeros((1, 8), jnp.float32).at[0, :3].set(fc2_b)

    return pl.pallas_call(
        _proj_kernel,
        grid=(S // RB,),
        in_specs=[
            pl.BlockSpec((RB, DIM), lambda i: (i, 0)),
            pl.BlockSpec((DIM, _PTOT), lambda i: (0, 0)),
            pl.BlockSpec((RB, NH * DK), lambda i: (i, 0)),
            pl.BlockSpec((RB, NH * DK), lambda i: (i, 0)),
            pl.BlockSpec((RB, G * DK), lambda i: (i, 0)),
            pl.BlockSpec((RB, G * DK), lambda i: (i, 0)),
            pl.BlockSpec((CPAD, RB), lambda i: (0, i)),
            pl.BlockSpec((HID, DK), lambda i: (0, 0)),
            pl.BlockSpec((1, HID), lambda i: (0, 0)),
            pl.BlockSpec((8, HID), lambda i: (0, 0)),
            pl.BlockSpec((1, 8), lambda i: (0, 0)),
        ],
        out_specs=[
            pl.BlockSpec((RB, NH * DK), lambda i: (i, 0)),
            pl.BlockSpec((RB, G * DK), lambda i: (i, 0)),
            pl.BlockSpec((RB, G * DV), lambda i: (i, 0)),
            pl.BlockSpec((RB, G * DK), lambda i: (i, 0)),
            pl.BlockSpec((RB, G * DV), lambda i: (i, 0)),
            pl.BlockSpec((CPAD, G * DK), lambda i: (0, 0)),
            pl.BlockSpec((CPAD, G * DV), lambda i: (0, 0)),
            pl.BlockSpec((RB, 8), lambda i: (i, 0)),
        ],
        out_shape=[
            jax.ShapeDtypeStruct((S, NH * DK), jnp.bfloat16),
            jax.ShapeDtypeStruct((S, G * DK), jnp.bfloat16),
            jax.ShapeDtypeStruct((S, G * DV), jnp.bfloat16),
            jax.ShapeDtypeStruct((S, G * DK), jnp.bfloat16),
            jax.ShapeDtypeStruct((S, G * DV), jnp.bfloat16),
            jax.ShapeDtypeStruct((CPAD, G * DK), jnp.float32),
            jax.ShapeDtypeStruct((CPAD, G * DV), jnp.float32),
            jax.ShapeDtypeStruct((S, 8), jnp.float32),
        ],
    )(x2.astype(jnp.bfloat16), w_all.astype(jnp.bfloat16),
      cosq, sinq, cosk, sink, pool,
      fc1_w.astype(jnp.bfloat16), fc1_b.reshape(1, HID),
      fc2w_pad.astype(jnp.bfloat16), fc2b_pad)


def _run_attn(qr, wg, ks, vs, kw, vw, kc, vc, W_out):
    _, ovp, expm = _const_tables()
    qr = qr.astype(jnp.bfloat16)
    ks = ks.astype(jnp.bfloat16)
    vs = vs.astype(jnp.bfloat16)
    kw = kw.astype(jnp.bfloat16)
    vw = vw.astype(jnp.bfloat16)
    kc = kc.astype(jnp.bfloat16)
    vc = vc.astype(jnp.bfloat16)
    ovp = ovp.astype(jnp.bfloat16)
    expm = expm.astype(jnp.bfloat16)
    wout = W_out.T.astype(jnp.bfloat16)

    ncalls = 4
    per = S // QB // ncalls
    parts = []
    for c in range(ncalls):
        w = (c + 1) * per * QB
        parts.append(pl.pallas_call(
            functools.partial(_attn_kernel, w, c * per),
            grid=(per,),
            in_specs=[
                pl.BlockSpec((QB, NH * DK), lambda i, c=c: (c * per + i, 0)),
                pl.BlockSpec((QB, 8), lambda i, c=c: (c * per + i, 0)),
                pl.BlockSpec((w, G * DK), lambda i: (0, 0)),
                pl.BlockSpec((w, G * DV), lambda i: (0, 0)),
                pl.BlockSpec((S, G * DK), lambda i: (0, 0)),
                pl.BlockSpec((S, G * DV), lambda i: (0, 0)),
                pl.BlockSpec((CPAD, G * DK), lambda i: (0, 0)),
                pl.BlockSpec((CPAD, G * DV), lambda i: (0, 0)),
                pl.BlockSpec((CPAD, NB), lambda i: (0, 0)),
                pl.BlockSpec((NB, w), lambda i: (0, 0)),
                pl.BlockSpec((NH * DV, DIM), lambda i: (0, 0)),
            ],
            out_specs=pl.BlockSpec((QB, DIM), lambda i: (i, 0)),
            out_shape=jax.ShapeDtypeStruct((per * QB, DIM), jnp.float32),
        )(qr, wg, ks[:w], vs[:w], kw, vw, kc, vc, ovp, expm[:, :w], wout))
    return jnp.concatenate(parts, axis=0)


@functools.partial(jax.jit, static_argnames=())
def kernel(x, W_Q, W_K_sel, W_V_sel, W_K_win, W_V_win, W_K_cmp, W_V_cmp,
           W_out, fc1_w, fc1_b, fc2_w, fc2_b):
    x2 = x.reshape(S, DIM)
    w_all = jnp.concatenate(
        [W_Q, W_K_sel, W_V_sel, W_K_win, W_V_win, W_K_cmp, W_V_cmp],
        axis=0).T
    qr, ks, vs, kw, vw, kc, vc, wg = _run_proj(
        x2, w_all, fc1_w, fc1_b, fc2_w, fc2_b)
    out = _run_attn(qr, wg, ks, vs, kw, vw, kc, vc, W_out)
    return out.reshape(B, S, DIM)


# speedup vs baseline: 3.2395x; 1.0007x over previous
import functools

import jax
import jax.numpy as jnp
import numpy as np
from jax.experimental import pallas as pl

B = 1
S = 2048
DIM = 768
NH = 12
G = 2
HPG = NH // G
DK = 64
DV = 64
CL = 32
CD = 16
LSEL = 64
NSEL = 8
WIN = 512
TAU = 1.0
HID = DK // 2
C = (S - CL) // CD + 1
CPAD = 128
NB = S // LSEL
QB = 512
NEG = -1e30
SCALE = 1.0 / 8.0

_OFF_Q = 0
_OFF_KS = NH * DK
_OFF_VS = _OFF_KS + G * DK
_OFF_KW = _OFF_VS + G * DV
_OFF_VW = _OFF_KW + G * DK
_OFF_KC = _OFF_VW + G * DV
_OFF_VC = _OFF_KC + G * DK
_PTOT = _OFF_VC + G * DV


def _swap_halves(xx, nheads):
    parts = []
    for h in range(nheads):
        base = h * DK
        parts.append(xx[:, base + DK // 2: base + DK])
        parts.append(xx[:, base: base + DK // 2])
    return jnp.concatenate(parts, axis=1)


RB = 512


def _bf16_dot(a, b, dn):
    return jax.lax.dot_general(a.astype(jnp.bfloat16), b.astype(jnp.bfloat16),
                               dn, preferred_element_type=jnp.float32)


def _proj_kernel(x_ref, w_ref, cosq_ref, sinq_ref, cosk_ref, sink_ref,
                 pool_ref, fc1w_ref, fc1b_ref, fc2w_ref, fc2b_ref,
                 qr_ref, ks_ref, vs_ref, kw_ref, vw_ref, kc_ref, vc_ref,
                 wg_ref):
    step = pl.program_id(0)
    x = x_ref[...]
    p = jax.lax.dot_general(x, w_ref[...], (((1,), (0,)), ((), ())),
                            preferred_element_type=jnp.float32)
    q = p[:, _OFF_Q:_OFF_KS]
    cq = cosq_ref[...]
    sq = sinq_ref[...]
    ck = cosk_ref[...]
    sk = sink_ref[...]
    qr_ref[...] = (q * cq + _swap_halves(q, NH) * sq).astype(jnp.bfloat16)

    ks = p[:, _OFF_KS:_OFF_VS]
    ks_ref[...] = (ks * ck + _swap_halves(ks, G) * sk).astype(jnp.bfloat16)
    vs_ref[...] = p[:, _OFF_VS:_OFF_KW].astype(jnp.bfloat16)
    kw = p[:, _OFF_KW:_OFF_VW]
    kw_ref[...] = (kw * ck + _swap_halves(kw, G) * sk).astype(jnp.bfloat16)
    vw_ref[...] = p[:, _OFF_VW:_OFF_KC].astype(jnp.bfloat16)
    kcr = p[:, _OFF_KC:_OFF_VC]
    kcr = kcr * ck + _swap_halves(kcr, G) * sk
    pool = pool_ref[...]
    kc_part = jax.lax.dot_general(pool, kcr, (((1,), (0,)), ((), ())),
                                  preferred_element_type=jnp.float32,
                                  precision=jax.lax.Precision.HIGHEST)
    vc_part = jax.lax.dot_general(pool, p[:, _OFF_VC:_PTOT],
                                  (((1,), (0,)), ((), ())),
                                  preferred_element_type=jnp.float32,
                                  precision=jax.lax.Precision.HIGHEST)

    @pl.when(step == 0)
    def _init():
        kc_ref[...] = kc_part
        vc_ref[...] = vc_part

    @pl.when(step > 0)
    def _acc():
        kc_ref[...] += kc_part
        vc_ref[...] += vc_part

    @pl.when(step == pl.num_programs(0) - 1)
    def _finish():
        kc_ref[...] = kc_ref[...] * (1.0 / CL)
        vc_ref[...] = vc_ref[...] * (1.0 / CL)

    fc1w = fc1w_ref[...]
    fc1b = fc1b_ref[...]
    fc2w = fc2w_ref[...]
    fc2b = fc2b_ref[...]
    gate_cols = []
    for g in range(G):
        qg = q[:, g * HPG * DK:(g + 1) * HPG * DK]
        acc = qg[:, 0:DK]
        for h in range(1, HPG):
            acc = acc + qg[:, h * DK:(h + 1) * DK]
        qgp = acc / float(HPG)
        h1 = _bf16_dot(qgp, fc1w, (((1,), (1,)), ((), ()))) + fc1b
        h1 = h1 * jax.nn.sigmoid(h1)
        gl = _bf16_dot(h1, fc2w, (((1,), (1,)), ((), ()))) + fc2b
        x0 = gl[:, 0:1]
        x1 = gl[:, 1:2]
        x2 = gl[:, 2:3]
        mx = jnp.maximum(jnp.maximum(x0, x1), x2)
        mn = jnp.minimum(jnp.minimum(x0, x1), x2)
        mid = x0 + x1 + x2 - mx - mn
        e0 = jnp.exp(x0 - mx)
        e1 = jnp.exp(x1 - mx)
        e2 = jnp.exp(x2 - mx)
        z = e0 + e1 + e2
        peaked = (mx - mid) > 50.0
        a0 = x0 == mx
        a1 = (x1 == mx) & (~a0)
        a2 = (x2 == mx) & (~a0) & (~a1)
        w0 = jnp.where(peaked, a0.astype(jnp.float32), e0 / z)
        w1 = jnp.where(peaked, a1.astype(jnp.float32), e1 / z)
        w2 = jnp.where(peaked, a2.astype(jnp.float32), e2 / z)
        gate_cols += [w0, w1, w2]
    gate_cols.append(jnp.zeros((x.shape[0], 2), jnp.float32))
    wg_ref[...] = jnp.concatenate(gate_cols, axis=1)


def _masked_softmax(scores, mask):
    sm = jnp.where(mask, scores, NEG)
    mx = jnp.max(sm, axis=-1, keepdims=True)
    p = jnp.where(mask, jnp.exp(sm - mx), 0.0)
    denom = jnp.sum(p, axis=-1, keepdims=True)
    return jnp.where(denom > 0.0, p / jnp.where(denom > 0.0, denom, 1.0), 0.0)


def _softmax_ne(scores, mask):
    sm = jnp.where(mask, scores, NEG)
    mx = jnp.max(sm, axis=-1, keepdims=True)
    p = jnp.exp(sm - mx)
    return p / jnp.sum(p, axis=-1, keepdims=True)


def _attn_kernel(w, qoff, qr_ref, wg_ref, ks_ref, vs_ref, kw_ref, vw_ref,
                 kc_ref, vc_ref, ov_ref, exp_ref, wout_ref, out_ref):
    i = qoff + pl.program_id(0)
    q0 = i * QB
    t = q0 + jax.lax.broadcasted_iota(jnp.int32, (QB, 1), 0)
    ccol = jax.lax.broadcasted_iota(jnp.int32, (QB, CPAD), 1)
    bcol = jax.lax.broadcasted_iota(jnp.int32, (QB, NB), 1)

    m_cmp = ((ccol * CD + CL) <= (t + 1)) & (ccol < C)
    causal_blk = (bcol * LSEL) <= t
    forced = (bcol == 0) | (bcol == (t // LSEL))

    o_cmp_all = []
    sel_all = []
    gate_all = []
    for g in range(G):
        kcg = kc_ref[:, g * DK:(g + 1) * DK]
        vcg = vc_ref[:, g * DV:(g + 1) * DV]

        o_cmps = []
        psum = None
        for h in range(HPG):
            qh = qr_ref[:, (g * HPG + h) * DK:(g * HPG + h + 1) * DK]
            sc = _bf16_dot(qh, kcg, (((1,), (1,)), ((), ())))
            pc = _masked_softmax(sc * SCALE, m_cmp)
            psum = pc if psum is None else psum + pc
            o_cmps.append(
                _bf16_dot(pc, vcg, (((1,), (0,)), ((), ()))))
        o_cmp_all.append(o_cmps)

        p_slc = _bf16_dot(psum, ov_ref[...], (((1,), (0,)), ((), ())))
        score = jnp.where(causal_blk, p_slc, -1e9) + \
            jnp.where(forced, 1e6, 0.0)

        sel = jnp.zeros((QB, NB), jnp.float32)
        work = score
        for _ in range(NSEL):
            mx = jnp.max(work, axis=-1, keepdims=True)
            cand = jnp.where(work == mx, bcol, NB + 1)
            amin = jnp.min(cand, axis=-1, keepdims=True)
            pick = bcol == amin
            sel = jnp.where(pick, 1.0, sel)
            work = jnp.where(pick, -3e9, work)
        sel_all.append(sel)
        gate_all.append((wg_ref[:, g * 3:g * 3 + 1],
                         wg_ref[:, g * 3 + 1:g * 3 + 2],
                         wg_ref[:, g * 3 + 2:g * 3 + 3]))

    WWIN = WIN + QB
    kcol = jax.lax.broadcasted_iota(jnp.int32, (QB, w), 1)
    causal = kcol <= t
    wstart = jnp.maximum(i - WIN // QB, 0) * QB
    jw = wstart + jax.lax.broadcasted_iota(jnp.int32, (QB, WWIN), 1)
    win_m = (jw <= t) & (jw > t - WIN)
    o_parts = []
    for g in range(G):
        tok = _bf16_dot(sel_all[g], exp_ref[...], (((1,), (0,)), ((), ())))
        sel_mask = (tok > 0.5) & causal
        ksg = ks_ref[:, g * DK:(g + 1) * DK]
        vsg = vs_ref[:, g * DV:(g + 1) * DV]
        kwg = kw_ref[pl.ds(wstart, WWIN), g * DK:(g + 1) * DK]
        vwg = vw_ref[pl.ds(wstart, WWIN), g * DV:(g + 1) * DV]
        w_cmp, w_sel, w_win = gate_all[g]
        for h in range(HPG):
            hh = g * HPG + h
            qh = qr_ref[:, hh * DK:(hh + 1) * DK]
            ss = _bf16_dot(qh, ksg, (((1,), (1,)), ((), ())))
            ps = _softmax_ne(ss * SCALE, sel_mask)
            o_sel = _bf16_dot(ps, vsg, (((1,), (0,)), ((), ())))
            sw = _bf16_dot(qh, kwg, (((1,), (1,)), ((), ())))
            pw = _softmax_ne(sw * SCALE, win_m)
            o_win = _bf16_dot(pw, vwg, (((1,), (0,)), ((), ())))
            o_parts.append(w_cmp * o_cmp_all[g][h] + w_sel * o_sel
                           + w_win * o_win)

    o_all = jnp.concatenate(o_parts, axis=1)
    out_ref[...] = _bf16_dot(o_all, wout_ref[...], (((1,), (0,)), ((), ())))


def _rope_tables():
    pos = jnp.arange(S, dtype=jnp.float32)
    half = DK // 2
    freqs = 1.0 / (10000.0 ** (jnp.arange(half, dtype=jnp.float32) / half))
    ang = pos[:, None] * freqs[None, :]
    cos = jnp.cos(ang)
    sin = jnp.sin(ang)
    cos_h = jnp.concatenate([cos, cos], axis=1)
    sin_h = jnp.concatenate([-sin, sin], axis=1)
    cosq = jnp.tile(cos_h, (1, NH))
    sinq = jnp.tile(sin_h, (1, NH))
    cosk = jnp.tile(cos_h, (1, G))
    sink = jnp.tile(sin_h, (1, G))
    return cosq, sinq, cosk, sink


def _const_tables():
    pool = np.zeros((CPAD, S), dtype=np.float32)
    for c in range(C):
        pool[c, c * CD:c * CD + CL] = 1.0
    pool = jnp.asarray(pool)

    cstart = np.arange(C) * CD
    bstart = np.arange(NB) * LSEL
    ov = np.clip(np.minimum(cstart[:, None] + CL, bstart[None, :] + LSEL)
                 - np.maximum(cstart[:, None], bstart[None, :]),
                 0, None).astype(np.float32) / CL
    ovp = np.zeros((CPAD, NB), dtype=np.float32)
    ovp[:C] = ov
    ovp = jnp.asarray(ovp)

    expm = np.zeros((NB, S), dtype=np.float32)
    for bnum in range(NB):
        expm[bnum, bnum * LSEL:(bnum + 1) * LSEL] = 1.0
    expm = jnp.asarray(expm)
    return pool, ovp, expm


def _run_proj(x2, w_all, fc1_w, fc1_b, fc2_w, fc2_b):
    cosq, sinq, cosk, sink = _rope_tables()
    pool, _, _ = _const_tables()
    fc2w_pad = jnp.zeros((8, HID), jnp.float32).at[:3].set(fc2_w)
    fc2b_pad = jnp.zeros((1, 8), jnp.float32).at[0, :3].set(fc2_b)

    return pl.pallas_call(
        _proj_kernel,
        grid=(S // RB,),
        in_specs=[
            pl.BlockSpec((RB, DIM), lambda i: (i, 0)),
            pl.BlockSpec((DIM, _PTOT), lambda i: (0, 0)),
            pl.BlockSpec((RB, NH * DK), lambda i: (i, 0)),
            pl.BlockSpec((RB, NH * DK), lambda i: (i, 0)),
            pl.BlockSpec((RB, G * DK), lambda i: (i, 0)),
            pl.BlockSpec((RB, G * DK), lambda i: (i, 0)),
            pl.BlockSpec((CPAD, RB), lambda i: (0, i)),
            pl.BlockSpec((HID, DK), lambda i: (0, 0)),
            pl.BlockSpec((1, HID), lambda i: (0, 0)),
            pl.BlockSpec((8, HID), lambda i: (0, 0)),
            pl.BlockSpec((1, 8), lambda i: (0, 0)),
        ],
        out_specs=[
            pl.BlockSpec((RB, NH * DK), lambda i: (i, 0)),
            pl.BlockSpec((RB, G * DK), lambda i: (i, 0)),
            pl.BlockSpec((RB, G * DV), lambda i: (i, 0)),
            pl.BlockSpec((RB, G * DK), lambda i: (i, 0)),
            pl.BlockSpec((RB, G * DV), lambda i: (i, 0)),
            pl.BlockSpec((CPAD, G * DK), lambda i: (0, 0)),
            pl.BlockSpec((CPAD, G * DV), lambda i: (0, 0)),
            pl.BlockSpec((RB, 8), lambda i: (i, 0)),
        ],
        out_shape=[
            jax.ShapeDtypeStruct((S, NH * DK), jnp.bfloat16),
            jax.ShapeDtypeStruct((S, G * DK), jnp.bfloat16),
            jax.ShapeDtypeStruct((S, G * DV), jnp.bfloat16),
            jax.ShapeDtypeStruct((S, G * DK), jnp.bfloat16),
            jax.ShapeDtypeStruct((S, G * DV), jnp.bfloat16),
            jax.ShapeDtypeStruct((CPAD, G * DK), jnp.float32),
            jax.ShapeDtypeStruct((CPAD, G * DV), jnp.float32),
            jax.ShapeDtypeStruct((S, 8), jnp.float32),
        ],
    )(x2.astype(jnp.bfloat16), w_all.astype(jnp.bfloat16),
      cosq, sinq, cosk, sink, pool,
      fc1_w.astype(jnp.bfloat16), fc1_b.reshape(1, HID),
      fc2w_pad.astype(jnp.bfloat16), fc2b_pad)


def _run_attn(qr, wg, ks, vs, kw, vw, kc, vc, W_out):
    _, ovp, expm = _const_tables()
    qr = qr.astype(jnp.bfloat16)
    ks = ks.astype(jnp.bfloat16)
    vs = vs.astype(jnp.bfloat16)
    kw = kw.astype(jnp.bfloat16)
    vw = vw.astype(jnp.bfloat16)
    kc = kc.astype(jnp.bfloat16)
    vc = vc.astype(jnp.bfloat16)
    ovp = ovp.astype(jnp.bfloat16)
    expm = expm.astype(jnp.bfloat16)
    wout = W_out.T.astype(jnp.bfloat16)

    ncalls = 4
    per = S // QB // ncalls
    parts = []
    for c in range(ncalls):
        w = (c + 1) * per * QB
        parts.append(pl.pallas_call(
            functools.partial(_attn_kernel, w, c * per),
            grid=(per,),
            in_specs=[
                pl.BlockSpec((QB, NH * DK), lambda i, c=c: (c * per + i, 0)),
                pl.BlockSpec((QB, 8), lambda i, c=c: (c * per + i, 0)),
                pl.BlockSpec((w, G * DK), lambda i: (0, 0)),
                pl.BlockSpec((w, G * DV), lambda i: (0, 0)),
                pl.BlockSpec((S, G * DK), lambda i: (0, 0)),
                pl.BlockSpec((S, G * DV), lambda i: (0, 0)),
                pl.BlockSpec((CPAD, G * DK), lambda i: (0, 0)),
                pl.BlockSpec((CPAD, G * DV), lambda i: (0, 0)),
                pl.BlockSpec((CPAD, NB), lambda i: (0, 0)),
                pl.BlockSpec((NB, w), lambda i: (0, 0)),
                pl.BlockSpec((NH * DV, DIM), lambda i: (0, 0)),
            ],
            out_specs=pl.BlockSpec((QB, DIM), lambda i: (i, 0)),
            out_shape=jax.ShapeDtypeStruct((per * QB, DIM), jnp.float32),
        )(qr, wg, ks[:w], vs[:w], kw, vw, kc, vc, ovp, expm[:, :w], wout))
    return jnp.concatenate(parts, axis=0)


@functools.partial(jax.jit, static_argnames=())
def kernel(x, W_Q, W_K_sel, W_V_sel, W_K_win, W_V_win, W_K_cmp, W_V_cmp,
           W_out, fc1_w, fc1_b, fc2_w, fc2_b):
    x2 = x.reshape(S, DIM)
    w_all = jnp.concatenate(
        [W_Q, W_K_sel, W_V_sel, W_K_win, W_V_win, W_K_cmp, W_V_cmp],
        axis=0).T
    qr, ks, vs, kw, vw, kc, vc, wg = _run_proj(
        x2, w_all, fc1_w, fc1_b, fc2_w, fc2_b)
    out = _run_attn(qr, wg, ks, vs, kw, vw, kc, vc, W_out)
    return out.reshape(B, S, DIM)
